# Initial kernel scaffold; baseline (speedup 1.0000x reference)
import jax
import jax.numpy as jnp
from jax.experimental import pallas as pl


def kernel(x, edge_index, edge_weight, W1, b1, W2, b2):
    raise NotImplementedError("write your pallas kernel here")


# speedup vs baseline: 1.1212x; 1.1212x over previous
import functools

import jax
import jax.numpy as jnp
from jax import lax
from jax.experimental import pallas as pl
from jax.experimental.pallas import tpu as pltpu
from jax.experimental.pallas import tpu_sc as plsc

N = 10000
E = 160000
D_IN = 256
D_H = 512
NC = 2
NS = 16
NW = NC * NS
FC = 128
NFC = D_H // FC
RB = 400
NRB = N // RB

EPT_G = E // NW
EPT_C = E // NS
CB = 80
NCB = EPT_C // CB
CS = 40
NCS = EPT_G // CS
RPT = N // NS

_mesh = plsc.VectorSubcoreMesh(core_axis_name="c", subcore_axis_name="s")

f32 = jnp.float32
i32 = jnp.int32


N_PAD = 10240
SLC = N_PAD // NS


@functools.partial(
    pl.kernel,
    out_type=jax.ShapeDtypeStruct((NC, N_PAD), f32),
    mesh=_mesh,
    compiler_params=pltpu.CompilerParams(needs_layout_passes=False),
    scratch_types=[
        pltpu.VMEM((N_PAD,), f32),
        pltpu.VMEM_SHARED((NS, N_PAD), f32),
        pltpu.VMEM((EPT_G + 16,), i32),
        pltpu.VMEM((EPT_G + 16,), f32),
        pltpu.VMEM((SLC,), f32),
        pltpu.VMEM((SLC,), f32),
    ],
)
def _deg_kernel(col_hbm, ew_hbm, out_hbm, acc, slots, colbuf, ewbuf,
                tmp, sumb):
    c = lax.axis_index("c")
    s = lax.axis_index("s")

    def zero_body(i, _):
        acc[pl.ds(i * 16, 16)] = jnp.zeros((16,), f32)
        return 0
    lax.fori_loop(0, N_PAD // 16, zero_body, 0)

    base = c * (E // NC) + s * EPT_G
    pltpu.sync_copy(col_hbm.at[pl.ds(base, EPT_G)], colbuf.at[pl.ds(0, EPT_G)])
    pltpu.sync_copy(ew_hbm.at[pl.ds(base, EPT_G)], ewbuf.at[pl.ds(0, EPT_G)])

    iota = lax.iota(i32, 16)
    ngroups = (EPT_G + 15) // 16

    def grp_body(g, _):
        off = g * 16
        m = (off + iota) < EPT_G
        cv = colbuf[pl.ds(off, 16)]
        wv = ewbuf[pl.ds(off, 16)]
        plsc.addupdate_scatter(acc, [cv], wv, mask=m)
        return 0
    lax.fori_loop(0, ngroups, grp_body, 0)

    pltpu.sync_copy(acc, slots.at[s])
    plsc.subcore_barrier()

    for p in range(NS):
        pltpu.sync_copy(slots.at[p, pl.ds(s * SLC, SLC)], tmp)
        for u in range(SLC // 16):
            sl = pl.ds(u * 16, 16)
            if p == 0:
                sumb[sl] = tmp[sl]
            else:
                sumb[sl] = sumb[sl] + tmp[sl]
    pltpu.sync_copy(sumb, out_hbm.at[c, pl.ds(s * SLC, SLC)])


@functools.partial(
    pl.kernel,
    out_type=jax.ShapeDtypeStruct((NFC * N, FC), f32),
    mesh=_mesh,
    compiler_params=pltpu.CompilerParams(needs_layout_passes=False),
    scratch_types=[
        pltpu.VMEM_SHARED((N, FC), f32),
        pltpu.VMEM((CB, FC), f32),
        pltpu.VMEM((CB,), i32),
        pltpu.VMEM((CB,), i32),
        pltpu.VMEM((CB,), i32),
        pltpu.VMEM((CB,), f32),
        pltpu.VMEM((16, FC), f32),
        pltpu.SemaphoreType.DMA,
    ],
)
def _prop_kernel(g_hbm, row_hbm, col_hbm, ew_hbm, out_hbm,
                 acc, rows, rowbuf, gidx, colbuf, ewv, zbuf, sem):
    c = lax.axis_index("c")
    s = lax.axis_index("s")
    NGT = N // 16
    NGL = (NGT + NS - 1) // NS

    def zb_body(r, _):
        for u in range(FC // 16):
            zbuf[r, pl.ds(u * 16, 16)] = jnp.zeros((16,), f32)
        return 0
    lax.fori_loop(0, 16, zb_body, 0)

    for j in range(NFC // NC):
        f = c * (NFC // NC) + j
        def zero_body(t, _):
            g = t * NS + s

            @pl.when(g < NGT)
            def _():
                pltpu.sync_copy(zbuf, acc.at[pl.ds(g * 16, 16)])
            return 0
        lax.fori_loop(0, NGL, zero_body, 0)
        plsc.subcore_barrier()

        def chunk_body(k, _):
            base = s * EPT_C + k * CB
            pltpu.sync_copy(row_hbm.at[pl.ds(base, CB)], rowbuf)
            pltpu.sync_copy(col_hbm.at[pl.ds(base, CB)], colbuf)
            pltpu.sync_copy(ew_hbm.at[pl.ds(base, CB)], ewv)
            foff = f * N
            for t in range(CB // 16):
                sl = pl.ds(t * 16, 16)
                gidx[sl] = rowbuf[sl] + foff
            pltpu.async_copy(g_hbm.at[gidx], rows, sem).wait()

            iota16 = lax.iota(i32, 16)
            for gi in range(CB // 16):
                vew = ewv[pl.ds(gi * 16, 16)]
                eidx = iota16 + gi * 16

                def pbody(p, _):
                    pidx = jnp.zeros((16,), i32) + p
                    v = plsc.load_gather(rows, [eidx, pidx])
                    plsc.store_scatter(rows, [eidx, pidx], v * vew)
                    return 0
                lax.fori_loop(0, FC, pbody, 0, unroll=8)

            pltpu.sync_copy(rows, acc.at[colbuf], add=True)
            return 0
        lax.fori_loop(0, NCB, chunk_body, 0)
        plsc.subcore_barrier()

        def wr_body(t, _):
            g = t * NS + s

            @pl.when(g < NGT)
            def _():
                pltpu.sync_copy(acc.at[pl.ds(g * 16, 16)],
                                out_hbm.at[pl.ds(f * N + g * 16, 16)])
            return 0
        lax.fori_loop(0, NGL, wr_body, 0)
        plsc.subcore_barrier()


@functools.partial(
    pl.kernel,
    out_type=jax.ShapeDtypeStruct((E,), f32),
    mesh=_mesh,
    compiler_params=pltpu.CompilerParams(needs_layout_passes=False),
    scratch_types=[
        pltpu.VMEM((CS,), i32),
        pltpu.VMEM((CS,), i32),
        pltpu.VMEM((CS, D_H), f32),
        pltpu.VMEM((CS, D_H), f32),
        pltpu.VMEM((48,), f32),
        pltpu.SemaphoreType.DMA,
        pltpu.SemaphoreType.DMA,
    ],
)
def _score_kernel(h_hbm, row_hbm, col_hbm, out_hbm,
                  rbuf, cbuf, hr, hc, sv, sem1, sem2):
    c = lax.axis_index("c")
    s = lax.axis_index("s")
    wid = s * NC + c
    ebase = wid * EPT_G
    iota16 = lax.iota(i32, 16)

    def chunk_body(k, _):
        base = ebase + k * CS
        pltpu.sync_copy(row_hbm.at[pl.ds(base, CS)], rbuf)
        pltpu.sync_copy(col_hbm.at[pl.ds(base, CS)], cbuf)
        cp1 = pltpu.async_copy(h_hbm.at[rbuf], hr, sem1)
        cp2 = pltpu.async_copy(h_hbm.at[cbuf], hc, sem2)
        cp1.wait()
        cp2.wait()

        for gi in range((CS + 15) // 16):
            nv = min(16, CS - gi * 16)

            def dot_body(i, outv):
                e = gi * 16 + i
                accv = hr[e, pl.ds(0, 16)] * hc[e, pl.ds(0, 16)]
                for u in range(1, D_H // 16):
                    sl = pl.ds(u * 16, 16)
                    accv = accv + hr[e, sl] * hc[e, sl]
                d = jnp.sum(accv)
                return jnp.where(iota16 == i, d, outv)
            outv = lax.fori_loop(0, nv, dot_body, jnp.zeros((16,), f32))
            sv[pl.ds(gi * 16, 16)] = 1.0 / (1.0 + jnp.exp(-outv))
        pltpu.sync_copy(sv.at[pl.ds(0, CS)], out_hbm.at[pl.ds(base, CS)])
        return 0
    lax.fori_loop(0, NCS, chunk_body, 0)


def _k1_body(x_ref, w1_ref, d0_ref, d1_ref, out_ref):
    deg = 1.0 + d0_ref[...] + d1_ref[...]
    dinv = lax.rsqrt(deg)
    h = jnp.dot(x_ref[...], w1_ref[...], preferred_element_type=f32)
    out_ref[...] = h * dinv


def _k1(x, W1, d0, d1):
    return pl.pallas_call(
        _k1_body,
        grid=(NRB, NFC),
        in_specs=[
            pl.BlockSpec((RB, D_IN), lambda i, f: (i, 0)),
            pl.BlockSpec((D_IN, FC), lambda i, f: (0, f)),
            pl.BlockSpec((RB, 1), lambda i, f: (i, 0)),
            pl.BlockSpec((RB, 1), lambda i, f: (i, 0)),
        ],
        out_specs=pl.BlockSpec((RB, FC), lambda i, f: (f * NRB + i, 0)),
        out_shape=jax.ShapeDtypeStruct((NFC * N, FC), f32),
    )(x, W1, d0, d1)


def _k2_body(s1_ref, g1_ref, d0_ref, d1_ref, w2_ref, b1_ref, out_ref,
             acc_ref):
    k = pl.program_id(2)
    deg = 1.0 + d0_ref[...] + d1_ref[...]
    dinv = lax.rsqrt(deg)
    z = jnp.maximum(dinv * (s1_ref[...] + g1_ref[...]) + b1_ref[0],
                    0.0)
    partial = jnp.dot(z, w2_ref[...], preferred_element_type=f32)

    @pl.when(k == 0)
    def _():
        acc_ref[...] = partial

    @pl.when(k != 0)
    def _():
        acc_ref[...] = acc_ref[...] + partial

    @pl.when(k == NFC - 1)
    def _():
        out_ref[...] = acc_ref[...] * dinv


def _k2(s1, g1, d0, d1, W2, b1r):
    return pl.pallas_call(
        _k2_body,
        grid=(NRB, NFC, NFC),
        in_specs=[
            pl.BlockSpec((RB, FC), lambda i, f, k: (k * NRB + i, 0)),
            pl.BlockSpec((RB, FC), lambda i, f, k: (k * NRB + i, 0)),
            pl.BlockSpec((RB, 1), lambda i, f, k: (i, 0)),
            pl.BlockSpec((RB, 1), lambda i, f, k: (i, 0)),
            pl.BlockSpec((FC, FC), lambda i, f, k: (k, f)),
            pl.BlockSpec((1, 1, FC), lambda i, f, k: (k, 0, 0)),
        ],
        out_specs=pl.BlockSpec((RB, FC), lambda i, f, k: (f * NRB + i, 0)),
        out_shape=jax.ShapeDtypeStruct((NFC * N, FC), f32),
        scratch_shapes=[pltpu.VMEM((RB, FC), f32)],
        compiler_params=pltpu.CompilerParams(
            dimension_semantics=("parallel", "parallel", "arbitrary")),
    )(s1, g1, d0, d1, W2, b1r)


def _k3_body(s2_ref, g2_ref, d0_ref, d1_ref, b2_ref, out_ref):
    deg = 1.0 + d0_ref[...] + d1_ref[...]
    dinv = lax.rsqrt(deg)
    out_ref[...] = dinv * (s2_ref[...] + g2_ref[...]) + b2_ref[0]


def _k3(s2, g2, d0, d1, b2r):
    return pl.pallas_call(
        _k3_body,
        grid=(NRB, NFC),
        in_specs=[
            pl.BlockSpec((RB, FC), lambda i, f: (f * NRB + i, 0)),
            pl.BlockSpec((RB, FC), lambda i, f: (f * NRB + i, 0)),
            pl.BlockSpec((RB, 1), lambda i, f: (i, 0)),
            pl.BlockSpec((RB, 1), lambda i, f: (i, 0)),
            pl.BlockSpec((1, 1, FC), lambda i, f: (f, 0, 0)),
        ],
        out_specs=pl.BlockSpec((RB, FC), lambda i, f: (i, f)),
        out_shape=jax.ShapeDtypeStruct((N, D_H), f32),
    )(s2, g2, d0, d1, b2r)


def kernel(x, edge_index, edge_weight, W1, b1, W2, b2):
    row = edge_index[0].astype(i32)
    col = edge_index[1].astype(i32)
    ew = edge_weight.astype(f32)
    b1r = b1.reshape(NFC, 1, FC)
    b2r = b2.reshape(NFC, 1, FC)

    deg_part = _deg_kernel(col, ew)
    d0 = deg_part[0, :N].reshape(N, 1)
    d1 = deg_part[1, :N].reshape(N, 1)
    g1 = _k1(x, W1, d0, d1)
    s1 = _prop_kernel(g1, row, col, ew)
    g2 = _k2(s1, g1, d0, d1, W2, b1r)
    s2 = _prop_kernel(g2, row, col, ew)
    h = _k3(s2, g2, d0, d1, b2r)
    return _score_kernel(h, row, col)


# speedup vs baseline: 1.3384x; 1.1937x over previous
import functools

import jax
import jax.numpy as jnp
from jax import lax
from jax.experimental import pallas as pl
from jax.experimental.pallas import tpu as pltpu
from jax.experimental.pallas import tpu_sc as plsc

N = 10000
E = 160000
D_IN = 256
D_H = 512
NC = 2
NS = 16
NW = NC * NS
FC = 128
NFC = D_H // FC
RB = 400
NRB = N // RB

EPT_G = E // NW
EPT_C = E // NS
CB = 80
NCB = EPT_C // CB
CS = 40
NCS = EPT_G // CS
RPT = N // NS

_mesh = plsc.VectorSubcoreMesh(core_axis_name="c", subcore_axis_name="s")

f32 = jnp.float32
i32 = jnp.int32


N_PAD = 10240
SLC = N_PAD // NS


@functools.partial(
    pl.kernel,
    out_type=jax.ShapeDtypeStruct((NC, N_PAD), f32),
    mesh=_mesh,
    compiler_params=pltpu.CompilerParams(needs_layout_passes=False),
    scratch_types=[
        pltpu.VMEM((N_PAD,), f32),
        pltpu.VMEM_SHARED((NS, N_PAD), f32),
        pltpu.VMEM((EPT_G + 16,), i32),
        pltpu.VMEM((EPT_G + 16,), f32),
        pltpu.VMEM((SLC,), f32),
        pltpu.VMEM((SLC,), f32),
    ],
)
def _deg_kernel(col_hbm, ew_hbm, out_hbm, acc, slots, colbuf, ewbuf,
                tmp, sumb):
    c = lax.axis_index("c")
    s = lax.axis_index("s")

    def zero_body(i, _):
        acc[pl.ds(i * 16, 16)] = jnp.zeros((16,), f32)
        return 0
    lax.fori_loop(0, N_PAD // 16, zero_body, 0)

    base = c * (E // NC) + s * EPT_G
    pltpu.sync_copy(col_hbm.at[pl.ds(base, EPT_G)], colbuf.at[pl.ds(0, EPT_G)])
    pltpu.sync_copy(ew_hbm.at[pl.ds(base, EPT_G)], ewbuf.at[pl.ds(0, EPT_G)])

    iota = lax.iota(i32, 16)
    ngroups = (EPT_G + 15) // 16

    def grp_body(g, _):
        off = g * 16
        m = (off + iota) < EPT_G
        cv = colbuf[pl.ds(off, 16)]
        wv = ewbuf[pl.ds(off, 16)]
        plsc.addupdate_scatter(acc, [cv], wv, mask=m)
        return 0
    lax.fori_loop(0, ngroups, grp_body, 0)

    pltpu.sync_copy(acc, slots.at[s])
    plsc.subcore_barrier()

    for p in range(NS):
        pltpu.sync_copy(slots.at[p, pl.ds(s * SLC, SLC)], tmp)
        for u in range(SLC // 16):
            sl = pl.ds(u * 16, 16)
            if p == 0:
                sumb[sl] = tmp[sl]
            else:
                sumb[sl] = sumb[sl] + tmp[sl]
    pltpu.sync_copy(sumb, out_hbm.at[c, pl.ds(s * SLC, SLC)])


@functools.partial(
    pl.kernel,
    out_type=jax.ShapeDtypeStruct((NFC * N, FC), f32),
    mesh=_mesh,
    compiler_params=pltpu.CompilerParams(needs_layout_passes=False),
    scratch_types=[
        pltpu.VMEM_SHARED((N, FC), f32),
        pltpu.VMEM((CB, FC), f32),
        pltpu.VMEM((CB, FC), f32),
        pltpu.VMEM((CB, FC), f32),
        pltpu.VMEM((CB, FC), f32),
        pltpu.VMEM((CB,), i32),
        pltpu.VMEM((CB,), i32),
        pltpu.VMEM((CB,), i32),
        pltpu.VMEM((CB,), i32),
        pltpu.VMEM((CB,), i32),
        pltpu.VMEM((CB,), i32),
        pltpu.VMEM((CB,), i32),
        pltpu.VMEM((CB,), i32),
        pltpu.VMEM((CB,), f32),
        pltpu.VMEM((CB,), f32),
        pltpu.VMEM((16, FC), f32),
        pltpu.SemaphoreType.DMA,
        pltpu.SemaphoreType.DMA,
        pltpu.SemaphoreType.DMA,
        pltpu.SemaphoreType.DMA,
        pltpu.SemaphoreType.DMA,
        pltpu.SemaphoreType.DMA,
    ],
)
def _prop_kernel(g_hbm, row_hbm, col_hbm, ew_hbm, out_hbm,
                 acc, rows0, rows1, sc0, sc1, gidx0, gidx1, cb0, cb1,
                 rw0, rw1, cl0, cl1, ew0, ew1, zbuf,
                 sem_g0, sem_g1, sem_s0, sem_s1, sem_m0, sem_m1):
    c = lax.axis_index("c")
    s = lax.axis_index("s")
    NGT = N // 16
    NGL = (NGT + NS - 1) // NS
    iota16 = lax.iota(i32, 16)

    def zb_body(r, _):
        for u in range(FC // 16):
            zbuf[r, pl.ds(u * 16, 16)] = jnp.zeros((16,), f32)
        return 0
    lax.fori_loop(0, 16, zb_body, 0)

    ebase = s * EPT_C
    rows_b = (rows0, rows1)
    sc_b = (sc0, sc1)
    gidx_b = (gidx0, gidx1)
    cb_b = (cb0, cb1)
    rw_b = (rw0, rw1)
    cl_b = (cl0, cl1)
    ew_b = (ew0, ew1)
    sem_g = (sem_g0, sem_g1)
    sem_s = (sem_s0, sem_s1)
    sem_m = (sem_m0, sem_m1)

    def meta_load(k, b):
        src = pl.ds(ebase + k * CB, CB)
        pltpu.async_copy(row_hbm.at[src], rw_b[b], sem_m[b])
        pltpu.async_copy(col_hbm.at[src], cl_b[b], sem_m[b])
        pltpu.async_copy(ew_hbm.at[src], ew_b[b], sem_m[b])

    def meta_wait(k, b):
        src = pl.ds(ebase + k * CB, CB)
        pltpu.make_async_copy(row_hbm.at[src], rw_b[b], sem_m[b]).wait()
        pltpu.make_async_copy(col_hbm.at[src], cl_b[b], sem_m[b]).wait()
        pltpu.make_async_copy(ew_hbm.at[src], ew_b[b], sem_m[b]).wait()

    for j in range(NFC // NC):
        f = c * (NFC // NC) + j
        foff = f * N

        def zero_body(t, _):
            g = t * NS + s

            @pl.when(g < NGT)
            def _():
                pltpu.sync_copy(zbuf, acc.at[pl.ds(g * 16, 16)])
            return 0
        lax.fori_loop(0, NGL, zero_body, 0)
        plsc.subcore_barrier()

        def stage_g(b):
            for g in range(CB // 16):
                sl = pl.ds(g * 16, 16)
                gidx_b[b][sl] = rw_b[b][sl] + foff

        def stage_c(b):
            for g in range(CB // 16):
                sl = pl.ds(g * 16, 16)
                cb_b[b][sl] = cl_b[b][sl]

        def gather(b):
            return pltpu.async_copy(g_hbm.at[gidx_b[b]], rows_b[b],
                                    sem_g[b])

        def scale(b):
            for gi in range(CB // 16):
                vew = ew_b[b][pl.ds(gi * 16, 16)]
                eidx = iota16 + gi * 16

                def pbody(p, _):
                    pidx = jnp.zeros((16,), i32) + p
                    v = plsc.load_gather(rows_b[b], [eidx, pidx])
                    plsc.store_scatter(sc_b[b], [eidx, pidx], v * vew)
                    return 0
                lax.fori_loop(0, FC, pbody, 0, unroll=8)

        def scatter(b):
            return pltpu.async_copy(sc_b[b], acc.at[cb_b[b]], sem_s[b],
                                    add=True)

        def wait_g(b):
            pltpu.make_async_copy(g_hbm.at[gidx_b[b]], rows_b[b],
                                  sem_g[b]).wait()

        def wait_s(b):
            pltpu.make_async_copy(sc_b[b], acc.at[cb_b[b]],
                                  sem_s[b]).wait()

        meta_load(0, 0)
        meta_wait(0, 0)
        stage_g(0)
        gather(0)
        meta_load(1, 1)

        def pair_body(kk, _):
            a = 2 * kk
            meta_wait(a + 1, 1)
            stage_g(1)
            gather(1)
            wait_g(0)

            @pl.when(kk > 0)
            def _():
                wait_s(0)
            scale(0)
            stage_c(0)
            scatter(0)
            meta_load(a + 2, 0)
            meta_wait(a + 2, 0)
            stage_g(0)
            gather(0)
            wait_g(1)

            @pl.when(kk > 0)
            def _():
                wait_s(1)
            scale(1)
            stage_c(1)
            scatter(1)
            meta_load(jnp.minimum(a + 3, NCB - 1), 1)
            return 0
        lax.fori_loop(0, (NCB - 1) // 2, pair_body, 0)

        wait_g(0)
        wait_s(0)
        scale(0)
        stage_c(0)
        scatter(0)
        meta_wait(NCB - 1, 1)
        wait_s(0)
        wait_s(1)
        plsc.subcore_barrier()

        def wr_body(t, _):
            g = t * NS + s

            @pl.when(g < NGT)
            def _():
                pltpu.sync_copy(acc.at[pl.ds(g * 16, 16)],
                                out_hbm.at[pl.ds(f * N + g * 16, 16)])
            return 0
        lax.fori_loop(0, NGL, wr_body, 0)
        plsc.subcore_barrier()


@functools.partial(
    pl.kernel,
    out_type=jax.ShapeDtypeStruct((E,), f32),
    mesh=_mesh,
    compiler_params=pltpu.CompilerParams(needs_layout_passes=False),
    scratch_types=[
        pltpu.VMEM((EPT_G,), i32),
        pltpu.VMEM((EPT_G,), i32),
        pltpu.VMEM((CS, D_H), f32),
        pltpu.VMEM((CS, D_H), f32),
        pltpu.VMEM((CS, D_H), f32),
        pltpu.VMEM((CS, D_H), f32),
        pltpu.VMEM((48,), f32),
        pltpu.SemaphoreType.DMA,
        pltpu.SemaphoreType.DMA,
        pltpu.SemaphoreType.DMA,
        pltpu.SemaphoreType.DMA,
    ],
)
def _score_kernel(h_hbm, row_hbm, col_hbm, out_hbm,
                  rall, call, hr0, hr1, hc0, hc1, sv,
                  semr0, semr1, semc0, semc1):
    c = lax.axis_index("c")
    s = lax.axis_index("s")
    wid = s * NC + c
    ebase = wid * EPT_G
    iota16 = lax.iota(i32, 16)

    pltpu.sync_copy(row_hbm.at[pl.ds(ebase, EPT_G)], rall)
    pltpu.sync_copy(col_hbm.at[pl.ds(ebase, EPT_G)], call)

    hr_b = (hr0, hr1)
    hc_b = (hc0, hc1)
    semr = (semr0, semr1)
    semc = (semc0, semc1)

    def gather(k, b):
        idx_r = rall.at[pl.ds(k * CS, CS)]
        idx_c = call.at[pl.ds(k * CS, CS)]
        pltpu.async_copy(h_hbm.at[idx_r], hr_b[b], semr[b])
        pltpu.async_copy(h_hbm.at[idx_c], hc_b[b], semc[b])

    def wait(k, b):
        idx_r = rall.at[pl.ds(k * CS, CS)]
        idx_c = call.at[pl.ds(k * CS, CS)]
        pltpu.make_async_copy(h_hbm.at[idx_r], hr_b[b], semr[b]).wait()
        pltpu.make_async_copy(h_hbm.at[idx_c], hc_b[b], semc[b]).wait()

    def compute(k, b):
        hr = hr_b[b]
        hc = hc_b[b]
        for gi in range((CS + 15) // 16):
            nv = min(16, CS - gi * 16)

            def dot_body(i, outv):
                e = gi * 16 + i
                accv = hr[e, pl.ds(0, 16)] * hc[e, pl.ds(0, 16)]
                for u in range(1, D_H // 16):
                    sl = pl.ds(u * 16, 16)
                    accv = accv + hr[e, sl] * hc[e, sl]
                d = jnp.sum(accv)
                return jnp.where(iota16 == i, d, outv)
            outv = lax.fori_loop(0, nv, dot_body, jnp.zeros((16,), f32))
            sv[pl.ds(gi * 16, 16)] = 1.0 / (1.0 + jnp.exp(-outv))
        pltpu.sync_copy(sv.at[pl.ds(0, CS)],
                        out_hbm.at[pl.ds(ebase + k * CS, CS)])

    gather(0, 0)

    def pair_body(kk, _):
        a = 2 * kk
        gather(a + 1, 1)
        wait(a, 0)
        compute(a, 0)
        gather(a + 2, 0)
        wait(a + 1, 1)
        compute(a + 1, 1)
        return 0
    lax.fori_loop(0, (NCS - 1) // 2, pair_body, 0)

    wait(NCS - 1, 0)
    compute(NCS - 1, 0)


def _k1_body(x_ref, w1_ref, d0_ref, d1_ref, out_ref):
    deg = 1.0 + d0_ref[...] + d1_ref[...]
    dinv = lax.rsqrt(deg)
    h = jnp.dot(x_ref[...], w1_ref[...], preferred_element_type=f32)
    out_ref[...] = h * dinv


def _k1(x, W1, d0, d1):
    return pl.pallas_call(
        _k1_body,
        grid=(NRB, NFC),
        in_specs=[
            pl.BlockSpec((RB, D_IN), lambda i, f: (i, 0)),
            pl.BlockSpec((D_IN, FC), lambda i, f: (0, f)),
            pl.BlockSpec((RB, 1), lambda i, f: (i, 0)),
            pl.BlockSpec((RB, 1), lambda i, f: (i, 0)),
        ],
        out_specs=pl.BlockSpec((RB, FC), lambda i, f: (f * NRB + i, 0)),
        out_shape=jax.ShapeDtypeStruct((NFC * N, FC), f32),
    )(x, W1, d0, d1)


def _k2_body(s1_ref, g1_ref, d0_ref, d1_ref, w2_ref, b1_ref, out_ref,
             acc_ref):
    k = pl.program_id(2)
    deg = 1.0 + d0_ref[...] + d1_ref[...]
    dinv = lax.rsqrt(deg)
    z = jnp.maximum(dinv * (s1_ref[...] + g1_ref[...]) + b1_ref[0],
                    0.0)
    partial = jnp.dot(z, w2_ref[...], preferred_element_type=f32)

    @pl.when(k == 0)
    def _():
        acc_ref[...] = partial

    @pl.when(k != 0)
    def _():
        acc_ref[...] = acc_ref[...] + partial

    @pl.when(k == NFC - 1)
    def _():
        out_ref[...] = acc_ref[...] * dinv


def _k2(s1, g1, d0, d1, W2, b1r):
    return pl.pallas_call(
        _k2_body,
        grid=(NRB, NFC, NFC),
        in_specs=[
            pl.BlockSpec((RB, FC), lambda i, f, k: (k * NRB + i, 0)),
            pl.BlockSpec((RB, FC), lambda i, f, k: (k * NRB + i, 0)),
            pl.BlockSpec((RB, 1), lambda i, f, k: (i, 0)),
            pl.BlockSpec((RB, 1), lambda i, f, k: (i, 0)),
            pl.BlockSpec((FC, FC), lambda i, f, k: (k, f)),
            pl.BlockSpec((1, 1, FC), lambda i, f, k: (k, 0, 0)),
        ],
        out_specs=pl.BlockSpec((RB, FC), lambda i, f, k: (f * NRB + i, 0)),
        out_shape=jax.ShapeDtypeStruct((NFC * N, FC), f32),
        scratch_shapes=[pltpu.VMEM((RB, FC), f32)],
        compiler_params=pltpu.CompilerParams(
            dimension_semantics=("parallel", "parallel", "arbitrary")),
    )(s1, g1, d0, d1, W2, b1r)


def _k3_body(s2_ref, g2_ref, d0_ref, d1_ref, b2_ref, out_ref):
    deg = 1.0 + d0_ref[...] + d1_ref[...]
    dinv = lax.rsqrt(deg)
    out_ref[...] = dinv * (s2_ref[...] + g2_ref[...]) + b2_ref[0]


def _k3(s2, g2, d0, d1, b2r):
    return pl.pallas_call(
        _k3_body,
        grid=(NRB, NFC),
        in_specs=[
            pl.BlockSpec((RB, FC), lambda i, f: (f * NRB + i, 0)),
            pl.BlockSpec((RB, FC), lambda i, f: (f * NRB + i, 0)),
            pl.BlockSpec((RB, 1), lambda i, f: (i, 0)),
            pl.BlockSpec((RB, 1), lambda i, f: (i, 0)),
            pl.BlockSpec((1, 1, FC), lambda i, f: (f, 0, 0)),
        ],
        out_specs=pl.BlockSpec((RB, FC), lambda i, f: (i, f)),
        out_shape=jax.ShapeDtypeStruct((N, D_H), f32),
    )(s2, g2, d0, d1, b2r)


def kernel(x, edge_index, edge_weight, W1, b1, W2, b2):
    row = edge_index[0].astype(i32)
    col = edge_index[1].astype(i32)
    ew = edge_weight.astype(f32)
    b1r = b1.reshape(NFC, 1, FC)
    b2r = b2.reshape(NFC, 1, FC)

    deg_part = _deg_kernel(col, ew)
    d0 = deg_part[0, :N].reshape(N, 1)
    d1 = deg_part[1, :N].reshape(N, 1)
    g1 = _k1(x, W1, d0, d1)
    s1 = _prop_kernel(g1, row, col, ew)
    g2 = _k2(s1, g1, d0, d1, W2, b1r)
    s2 = _prop_kernel(g2, row, col, ew)
    h = _k3(s2, g2, d0, d1, b2r)
    return _score_kernel(h, row, col)


# speedup vs baseline: 6.4389x; 4.8107x over previous
import functools

import jax
import jax.numpy as jnp
from jax import lax
from jax.experimental import pallas as pl
from jax.experimental.pallas import tpu as pltpu
from jax.experimental.pallas import tpu_sc as plsc

N = 10000
E = 160000
D_IN = 256
D_H = 512
NC = 2
NS = 16
NW = NC * NS
FC = 128
NFC = D_H // FC
RB = 400
NRB = N // RB

EPT_G = E // NW
EPT_C = E // NS
CB = 80
NCB = EPT_C // CB
CS = 40
NCS = EPT_G // CS
RPT = N // NS

_mesh = plsc.VectorSubcoreMesh(core_axis_name="c", subcore_axis_name="s")

f32 = jnp.float32
i32 = jnp.int32


N_PAD = 10240
SLC = N_PAD // NS


@functools.partial(
    pl.kernel,
    out_type=jax.ShapeDtypeStruct((NC, N_PAD), f32),
    mesh=_mesh,
    compiler_params=pltpu.CompilerParams(needs_layout_passes=False),
    scratch_types=[
        pltpu.VMEM((N_PAD,), f32),
        pltpu.VMEM_SHARED((NS, N_PAD), f32),
        pltpu.VMEM((EPT_G + 16,), i32),
        pltpu.VMEM((EPT_G + 16,), f32),
        pltpu.VMEM((SLC,), f32),
        pltpu.VMEM((SLC,), f32),
    ],
)
def _deg_kernel(col_hbm, ew_hbm, out_hbm, acc, slots, colbuf, ewbuf,
                tmp, sumb):
    c = lax.axis_index("c")
    s = lax.axis_index("s")

    def zero_body(i, _):
        acc[pl.ds(i * 16, 16)] = jnp.zeros((16,), f32)
        return 0
    lax.fori_loop(0, N_PAD // 16, zero_body, 0)

    base = c * (E // NC) + s * EPT_G
    pltpu.sync_copy(col_hbm.at[pl.ds(base, EPT_G)], colbuf.at[pl.ds(0, EPT_G)])
    pltpu.sync_copy(ew_hbm.at[pl.ds(base, EPT_G)], ewbuf.at[pl.ds(0, EPT_G)])

    iota = lax.iota(i32, 16)
    ngroups = (EPT_G + 15) // 16

    def grp_body(g, _):
        off = g * 16
        m = (off + iota) < EPT_G
        cv = colbuf[pl.ds(off, 16)]
        wv = ewbuf[pl.ds(off, 16)]
        plsc.addupdate_scatter(acc, [cv], wv, mask=m)
        return 0
    lax.fori_loop(0, ngroups, grp_body, 0)

    pltpu.sync_copy(acc, slots.at[s])
    plsc.subcore_barrier()

    for p in range(NS):
        pltpu.sync_copy(slots.at[p, pl.ds(s * SLC, SLC)], tmp)
        for u in range(SLC // 16):
            sl = pl.ds(u * 16, 16)
            if p == 0:
                sumb[sl] = tmp[sl]
            else:
                sumb[sl] = sumb[sl] + tmp[sl]
    pltpu.sync_copy(sumb, out_hbm.at[c, pl.ds(s * SLC, SLC)])


@functools.partial(
    pl.kernel,
    out_type=jax.ShapeDtypeStruct((NFC * N, FC), f32),
    mesh=_mesh,
    compiler_params=pltpu.CompilerParams(needs_layout_passes=False),
    scratch_types=[
        pltpu.VMEM_SHARED((N, FC), f32),
        pltpu.VMEM((CB, FC), f32),
        pltpu.VMEM((CB, FC), f32),
        pltpu.VMEM((CB, FC), f32),
        pltpu.VMEM((CB, FC), f32),
        pltpu.VMEM((CB,), i32),
        pltpu.VMEM((CB,), i32),
        pltpu.VMEM((CB,), i32),
        pltpu.VMEM((CB,), i32),
        pltpu.VMEM((CB,), i32),
        pltpu.VMEM((CB,), i32),
        pltpu.VMEM((CB,), i32),
        pltpu.VMEM((CB,), i32),
        pltpu.VMEM((CB,), f32),
        pltpu.VMEM((CB,), f32),
        pltpu.VMEM((16, FC), f32),
        pltpu.SemaphoreType.DMA,
        pltpu.SemaphoreType.DMA,
        pltpu.SemaphoreType.DMA,
        pltpu.SemaphoreType.DMA,
        pltpu.SemaphoreType.DMA,
        pltpu.SemaphoreType.DMA,
    ],
)
def _prop_kernel(g_hbm, row_hbm, col_hbm, ew_hbm, out_hbm,
                 acc, rows0, rows1, sc0, sc1, gidx0, gidx1, cb0, cb1,
                 rw0, rw1, cl0, cl1, ew0, ew1, zbuf,
                 sem_g0, sem_g1, sem_s0, sem_s1, sem_m0, sem_m1):
    c = lax.axis_index("c")
    s = lax.axis_index("s")
    NGT = N // 16
    NGL = (NGT + NS - 1) // NS
    iota16 = lax.iota(i32, 16)

    def zb_body(r, _):
        for u in range(FC // 16):
            zbuf[r, pl.ds(u * 16, 16)] = jnp.zeros((16,), f32)
        return 0
    lax.fori_loop(0, 16, zb_body, 0)

    ebase = s * EPT_C
    rows_b = (rows0, rows1)
    sc_b = (sc0, sc1)
    gidx_b = (gidx0, gidx1)
    cb_b = (cb0, cb1)
    rw_b = (rw0, rw1)
    cl_b = (cl0, cl1)
    ew_b = (ew0, ew1)
    sem_g = (sem_g0, sem_g1)
    sem_s = (sem_s0, sem_s1)
    sem_m = (sem_m0, sem_m1)

    def meta_load(k, b):
        src = pl.ds(ebase + k * CB, CB)
        pltpu.async_copy(row_hbm.at[src], rw_b[b], sem_m[b])
        pltpu.async_copy(col_hbm.at[src], cl_b[b], sem_m[b])
        pltpu.async_copy(ew_hbm.at[src], ew_b[b], sem_m[b])

    def meta_wait(k, b):
        src = pl.ds(ebase + k * CB, CB)
        pltpu.make_async_copy(row_hbm.at[src], rw_b[b], sem_m[b]).wait()
        pltpu.make_async_copy(col_hbm.at[src], cl_b[b], sem_m[b]).wait()
        pltpu.make_async_copy(ew_hbm.at[src], ew_b[b], sem_m[b]).wait()

    for j in range(NFC // NC):
        f = c * (NFC // NC) + j
        foff = f * N

        def zero_body(t, _):
            g = t * NS + s

            @pl.when(g < NGT)
            def _():
                pltpu.sync_copy(zbuf, acc.at[pl.ds(g * 16, 16)])
            return 0
        lax.fori_loop(0, NGL, zero_body, 0)
        plsc.subcore_barrier()

        def stage_g(b):
            for g in range(CB // 16):
                sl = pl.ds(g * 16, 16)
                gidx_b[b][sl] = rw_b[b][sl] + foff

        def stage_c(b):
            for g in range(CB // 16):
                sl = pl.ds(g * 16, 16)
                cb_b[b][sl] = cl_b[b][sl]

        def gather(b):
            return pltpu.async_copy(g_hbm.at[gidx_b[b]], rows_b[b],
                                    sem_g[b])

        def scale(b):
            zeros16 = jnp.zeros((16,), i32)

            def grp_body(gi, _):
                for l in range(16):
                    e = gi * 16 + l
                    wv = plsc.load_gather(ew_b[b], [zeros16 + e])
                    for u in range(FC // 16):
                        sl = pl.ds(u * 16, 16)
                        sc_b[b][e, sl] = rows_b[b][e, sl] * wv
                return 0
            lax.fori_loop(0, CB // 16, grp_body, 0)

        def scatter(b):
            return pltpu.async_copy(sc_b[b], acc.at[cb_b[b]], sem_s[b],
                                    add=True)

        def wait_g(b):
            pltpu.make_async_copy(g_hbm.at[gidx_b[b]], rows_b[b],
                                  sem_g[b]).wait()

        def wait_s(b):
            pltpu.make_async_copy(sc_b[b], acc.at[cb_b[b]],
                                  sem_s[b]).wait()

        meta_load(0, 0)
        meta_wait(0, 0)
        stage_g(0)
        gather(0)
        meta_load(1, 1)

        def pair_body(kk, _):
            a = 2 * kk
            meta_wait(a + 1, 1)
            stage_g(1)
            gather(1)
            wait_g(0)

            @pl.when(kk > 0)
            def _():
                wait_s(0)
            scale(0)
            stage_c(0)
            scatter(0)
            meta_load(a + 2, 0)
            meta_wait(a + 2, 0)
            stage_g(0)
            gather(0)
            wait_g(1)

            @pl.when(kk > 0)
            def _():
                wait_s(1)
            scale(1)
            stage_c(1)
            scatter(1)
            meta_load(jnp.minimum(a + 3, NCB - 1), 1)
            return 0
        lax.fori_loop(0, (NCB - 1) // 2, pair_body, 0)

        wait_g(0)
        wait_s(0)
        scale(0)
        stage_c(0)
        scatter(0)
        meta_wait(NCB - 1, 1)
        wait_s(0)
        wait_s(1)
        plsc.subcore_barrier()

        def wr_body(t, _):
            g = t * NS + s

            @pl.when(g < NGT)
            def _():
                pltpu.sync_copy(acc.at[pl.ds(g * 16, 16)],
                                out_hbm.at[pl.ds(f * N + g * 16, 16)])
            return 0
        lax.fori_loop(0, NGL, wr_body, 0)
        plsc.subcore_barrier()


@functools.partial(
    pl.kernel,
    out_type=jax.ShapeDtypeStruct((E,), f32),
    mesh=_mesh,
    compiler_params=pltpu.CompilerParams(needs_layout_passes=False),
    scratch_types=[
        pltpu.VMEM((EPT_G,), i32),
        pltpu.VMEM((EPT_G,), i32),
        pltpu.VMEM((CS, D_H), f32),
        pltpu.VMEM((CS, D_H), f32),
        pltpu.VMEM((CS, D_H), f32),
        pltpu.VMEM((CS, D_H), f32),
        pltpu.VMEM((48,), f32),
        pltpu.SemaphoreType.DMA,
        pltpu.SemaphoreType.DMA,
        pltpu.SemaphoreType.DMA,
        pltpu.SemaphoreType.DMA,
    ],
)
def _score_kernel(h_hbm, row_hbm, col_hbm, out_hbm,
                  rall, call, hr0, hr1, hc0, hc1, sv,
                  semr0, semr1, semc0, semc1):
    c = lax.axis_index("c")
    s = lax.axis_index("s")
    wid = s * NC + c
    ebase = wid * EPT_G
    iota16 = lax.iota(i32, 16)

    pltpu.sync_copy(row_hbm.at[pl.ds(ebase, EPT_G)], rall)
    pltpu.sync_copy(col_hbm.at[pl.ds(ebase, EPT_G)], call)

    hr_b = (hr0, hr1)
    hc_b = (hc0, hc1)
    semr = (semr0, semr1)
    semc = (semc0, semc1)

    def gather(k, b):
        idx_r = rall.at[pl.ds(k * CS, CS)]
        idx_c = call.at[pl.ds(k * CS, CS)]
        pltpu.async_copy(h_hbm.at[idx_r], hr_b[b], semr[b])
        pltpu.async_copy(h_hbm.at[idx_c], hc_b[b], semc[b])

    def wait(k, b):
        idx_r = rall.at[pl.ds(k * CS, CS)]
        idx_c = call.at[pl.ds(k * CS, CS)]
        pltpu.make_async_copy(h_hbm.at[idx_r], hr_b[b], semr[b]).wait()
        pltpu.make_async_copy(h_hbm.at[idx_c], hc_b[b], semc[b]).wait()

    def compute(k, b):
        hr = hr_b[b]
        hc = hc_b[b]
        for gi in range((CS + 15) // 16):
            nv = min(16, CS - gi * 16)

            def dot_body(i, outv):
                e = gi * 16 + i
                accv = hr[e, pl.ds(0, 16)] * hc[e, pl.ds(0, 16)]
                for u in range(1, D_H // 16):
                    sl = pl.ds(u * 16, 16)
                    accv = accv + hr[e, sl] * hc[e, sl]
                d = jnp.sum(accv)
                return jnp.where(iota16 == i, d, outv)
            outv = lax.fori_loop(0, nv, dot_body, jnp.zeros((16,), f32))
            sv[pl.ds(gi * 16, 16)] = 1.0 / (1.0 + jnp.exp(-outv))
        pltpu.sync_copy(sv.at[pl.ds(0, CS)],
                        out_hbm.at[pl.ds(ebase + k * CS, CS)])

    gather(0, 0)

    def pair_body(kk, _):
        a = 2 * kk
        gather(a + 1, 1)
        wait(a, 0)
        compute(a, 0)
        gather(a + 2, 0)
        wait(a + 1, 1)
        compute(a + 1, 1)
        return 0
    lax.fori_loop(0, (NCS - 1) // 2, pair_body, 0)

    wait(NCS - 1, 0)
    compute(NCS - 1, 0)


def _k1_body(x_ref, w1_ref, d0_ref, d1_ref, out_ref):
    deg = 1.0 + d0_ref[...] + d1_ref[...]
    dinv = lax.rsqrt(deg)
    h = jnp.dot(x_ref[...], w1_ref[...], preferred_element_type=f32)
    out_ref[...] = h * dinv


def _k1(x, W1, d0, d1):
    return pl.pallas_call(
        _k1_body,
        grid=(NRB, NFC),
        in_specs=[
            pl.BlockSpec((RB, D_IN), lambda i, f: (i, 0)),
            pl.BlockSpec((D_IN, FC), lambda i, f: (0, f)),
            pl.BlockSpec((RB, 1), lambda i, f: (i, 0)),
            pl.BlockSpec((RB, 1), lambda i, f: (i, 0)),
        ],
        out_specs=pl.BlockSpec((RB, FC), lambda i, f: (f * NRB + i, 0)),
        out_shape=jax.ShapeDtypeStruct((NFC * N, FC), f32),
    )(x, W1, d0, d1)


def _k2_body(s1_ref, g1_ref, d0_ref, d1_ref, w2_ref, b1_ref, out_ref,
             acc_ref):
    k = pl.program_id(2)
    deg = 1.0 + d0_ref[...] + d1_ref[...]
    dinv = lax.rsqrt(deg)
    z = jnp.maximum(dinv * (s1_ref[...] + g1_ref[...]) + b1_ref[0],
                    0.0)
    partial = jnp.dot(z, w2_ref[...], preferred_element_type=f32)

    @pl.when(k == 0)
    def _():
        acc_ref[...] = partial

    @pl.when(k != 0)
    def _():
        acc_ref[...] = acc_ref[...] + partial

    @pl.when(k == NFC - 1)
    def _():
        out_ref[...] = acc_ref[...] * dinv


def _k2(s1, g1, d0, d1, W2, b1r):
    return pl.pallas_call(
        _k2_body,
        grid=(NRB, NFC, NFC),
        in_specs=[
            pl.BlockSpec((RB, FC), lambda i, f, k: (k * NRB + i, 0)),
            pl.BlockSpec((RB, FC), lambda i, f, k: (k * NRB + i, 0)),
            pl.BlockSpec((RB, 1), lambda i, f, k: (i, 0)),
            pl.BlockSpec((RB, 1), lambda i, f, k: (i, 0)),
            pl.BlockSpec((FC, FC), lambda i, f, k: (k, f)),
            pl.BlockSpec((1, 1, FC), lambda i, f, k: (k, 0, 0)),
        ],
        out_specs=pl.BlockSpec((RB, FC), lambda i, f, k: (f * NRB + i, 0)),
        out_shape=jax.ShapeDtypeStruct((NFC * N, FC), f32),
        scratch_shapes=[pltpu.VMEM((RB, FC), f32)],
        compiler_params=pltpu.CompilerParams(
            dimension_semantics=("parallel", "parallel", "arbitrary")),
    )(s1, g1, d0, d1, W2, b1r)


def _k3_body(s2_ref, g2_ref, d0_ref, d1_ref, b2_ref, out_ref):
    deg = 1.0 + d0_ref[...] + d1_ref[...]
    dinv = lax.rsqrt(deg)
    out_ref[...] = dinv * (s2_ref[...] + g2_ref[...]) + b2_ref[0]


def _k3(s2, g2, d0, d1, b2r):
    return pl.pallas_call(
        _k3_body,
        grid=(NRB, NFC),
        in_specs=[
            pl.BlockSpec((RB, FC), lambda i, f: (f * NRB + i, 0)),
            pl.BlockSpec((RB, FC), lambda i, f: (f * NRB + i, 0)),
            pl.BlockSpec((RB, 1), lambda i, f: (i, 0)),
            pl.BlockSpec((RB, 1), lambda i, f: (i, 0)),
            pl.BlockSpec((1, 1, FC), lambda i, f: (f, 0, 0)),
        ],
        out_specs=pl.BlockSpec((RB, FC), lambda i, f: (i, f)),
        out_shape=jax.ShapeDtypeStruct((N, D_H), f32),
    )(s2, g2, d0, d1, b2r)


def kernel(x, edge_index, edge_weight, W1, b1, W2, b2):
    row = edge_index[0].astype(i32)
    col = edge_index[1].astype(i32)
    ew = edge_weight.astype(f32)
    b1r = b1.reshape(NFC, 1, FC)
    b2r = b2.reshape(NFC, 1, FC)

    deg_part = _deg_kernel(col, ew)
    d0 = deg_part[0, :N].reshape(N, 1)
    d1 = deg_part[1, :N].reshape(N, 1)
    g1 = _k1(x, W1, d0, d1)
    s1 = _prop_kernel(g1, row, col, ew)
    g2 = _k2(s1, g1, d0, d1, W2, b1r)
    s2 = _prop_kernel(g2, row, col, ew)
    h = _k3(s2, g2, d0, d1, b2r)
    return _score_kernel(h, row, col)


# speedup vs baseline: 7.3600x; 1.1431x over previous
import functools

import jax
import jax.numpy as jnp
from jax import lax
from jax.experimental import pallas as pl
from jax.experimental.pallas import tpu as pltpu
from jax.experimental.pallas import tpu_sc as plsc

N = 10000
E = 160000
D_IN = 256
D_H = 512
NC = 2
NS = 16
NW = NC * NS
FC = 128
NFC = D_H // FC
RB = 400
NRB = N // RB

EPT_G = E // NW
EPT_C = E // NS
CB = 80
NCB = EPT_C // CB
CS = 40
NCS = EPT_G // CS
RPT = N // NS

_mesh = plsc.VectorSubcoreMesh(core_axis_name="c", subcore_axis_name="s")

f32 = jnp.float32
i32 = jnp.int32


N_PAD = 10240
SLC = N_PAD // NS


@functools.partial(
    pl.kernel,
    out_type=jax.ShapeDtypeStruct((NC, N_PAD), f32),
    mesh=_mesh,
    compiler_params=pltpu.CompilerParams(needs_layout_passes=False),
    scratch_types=[
        pltpu.VMEM((N_PAD,), f32),
        pltpu.VMEM_SHARED((NS, N_PAD), f32),
        pltpu.VMEM((EPT_G + 16,), i32),
        pltpu.VMEM((EPT_G + 16,), f32),
        pltpu.VMEM((SLC,), f32),
        pltpu.VMEM((SLC,), f32),
    ],
)
def _deg_kernel(col_hbm, ew_hbm, out_hbm, acc, slots, colbuf, ewbuf,
                tmp, sumb):
    c = lax.axis_index("c")
    s = lax.axis_index("s")

    def zero_body(i, _):
        acc[pl.ds(i * 16, 16)] = jnp.zeros((16,), f32)
        return 0
    lax.fori_loop(0, N_PAD // 16, zero_body, 0)

    base = c * (E // NC) + s * EPT_G
    pltpu.sync_copy(col_hbm.at[pl.ds(base, EPT_G)], colbuf.at[pl.ds(0, EPT_G)])
    pltpu.sync_copy(ew_hbm.at[pl.ds(base, EPT_G)], ewbuf.at[pl.ds(0, EPT_G)])

    iota = lax.iota(i32, 16)
    ngroups = (EPT_G + 15) // 16

    def grp_body(g, _):
        off = g * 16
        m = (off + iota) < EPT_G
        cv = colbuf[pl.ds(off, 16)]
        wv = ewbuf[pl.ds(off, 16)]
        plsc.addupdate_scatter(acc, [cv], wv, mask=m)
        return 0
    lax.fori_loop(0, ngroups, grp_body, 0)

    pltpu.sync_copy(acc, slots.at[s])
    plsc.subcore_barrier()

    for p in range(NS):
        pltpu.sync_copy(slots.at[p, pl.ds(s * SLC, SLC)], tmp)
        for u in range(SLC // 16):
            sl = pl.ds(u * 16, 16)
            if p == 0:
                sumb[sl] = tmp[sl]
            else:
                sumb[sl] = sumb[sl] + tmp[sl]
    pltpu.sync_copy(sumb, out_hbm.at[c, pl.ds(s * SLC, SLC)])


@functools.partial(
    pl.kernel,
    out_type=jax.ShapeDtypeStruct((NFC * N, FC), f32),
    mesh=_mesh,
    compiler_params=pltpu.CompilerParams(needs_layout_passes=False),
    scratch_types=[
        pltpu.VMEM_SHARED((N, FC), f32),
        pltpu.VMEM((CB, FC), f32),
        pltpu.VMEM((CB, FC), f32),
        pltpu.VMEM((CB, FC), f32),
        pltpu.VMEM((CB, FC), f32),
        pltpu.VMEM((CB,), i32),
        pltpu.VMEM((CB,), i32),
        pltpu.VMEM((CB,), i32),
        pltpu.VMEM((CB,), i32),
        pltpu.VMEM((CB,), i32),
        pltpu.VMEM((CB,), i32),
        pltpu.VMEM((CB,), i32),
        pltpu.VMEM((CB,), i32),
        pltpu.VMEM((CB,), f32),
        pltpu.VMEM((CB,), f32),
        pltpu.VMEM((16, FC), f32),
        pltpu.SemaphoreType.DMA,
        pltpu.SemaphoreType.DMA,
        pltpu.SemaphoreType.DMA,
        pltpu.SemaphoreType.DMA,
        pltpu.SemaphoreType.DMA,
        pltpu.SemaphoreType.DMA,
    ],
)
def _prop_kernel(g_hbm, row_hbm, col_hbm, ew_hbm, out_hbm,
                 acc, rows0, rows1, sc0, sc1, gidx0, gidx1, cb0, cb1,
                 rw0, rw1, cl0, cl1, ew0, ew1, zbuf,
                 sem_g0, sem_g1, sem_s0, sem_s1, sem_m0, sem_m1):
    c = lax.axis_index("c")
    s = lax.axis_index("s")
    NGT = N // 16
    NGL = (NGT + NS - 1) // NS
    iota16 = lax.iota(i32, 16)

    def zb_body(r, _):
        for u in range(FC // 16):
            zbuf[r, pl.ds(u * 16, 16)] = jnp.zeros((16,), f32)
        return 0
    lax.fori_loop(0, 16, zb_body, 0)

    ebase = s * EPT_C
    rows_b = (rows0, rows1)
    sc_b = (sc0, sc1)
    gidx_b = (gidx0, gidx1)
    cb_b = (cb0, cb1)
    rw_b = (rw0, rw1)
    cl_b = (cl0, cl1)
    ew_b = (ew0, ew1)
    sem_g = (sem_g0, sem_g1)
    sem_s = (sem_s0, sem_s1)
    sem_m = (sem_m0, sem_m1)

    def meta_load(k, b):
        src = pl.ds(ebase + k * CB, CB)
        pltpu.async_copy(row_hbm.at[src], rw_b[b], sem_m[b])
        pltpu.async_copy(col_hbm.at[src], cl_b[b], sem_m[b])
        pltpu.async_copy(ew_hbm.at[src], ew_b[b], sem_m[b])

    def meta_wait(k, b):
        src = pl.ds(ebase + k * CB, CB)
        pltpu.make_async_copy(row_hbm.at[src], rw_b[b], sem_m[b]).wait()
        pltpu.make_async_copy(col_hbm.at[src], cl_b[b], sem_m[b]).wait()
        pltpu.make_async_copy(ew_hbm.at[src], ew_b[b], sem_m[b]).wait()

    for j in range(NFC // NC):
        f = c * (NFC // NC) + j
        foff = f * N

        def zero_body(t, _):
            g = t * NS + s

            @pl.when(g < NGT)
            def _():
                pltpu.sync_copy(zbuf, acc.at[pl.ds(g * 16, 16)])
            return 0
        lax.fori_loop(0, NGL, zero_body, 0)
        plsc.subcore_barrier()

        def stage_g(b):
            for g in range(CB // 16):
                sl = pl.ds(g * 16, 16)
                gidx_b[b][sl] = rw_b[b][sl] + foff

        def stage_c(b):
            for g in range(CB // 16):
                sl = pl.ds(g * 16, 16)
                cb_b[b][sl] = cl_b[b][sl]

        def gather(b):
            return pltpu.async_copy(g_hbm.at[gidx_b[b]], rows_b[b],
                                    sem_g[b])

        def scale(b):
            zeros16 = jnp.zeros((16,), i32)

            def grp_body(gi, _):
                for l in range(16):
                    e = gi * 16 + l
                    wv = plsc.load_gather(ew_b[b], [zeros16 + e])
                    for u in range(FC // 16):
                        sl = pl.ds(u * 16, 16)
                        sc_b[b][e, sl] = rows_b[b][e, sl] * wv
                return 0
            lax.fori_loop(0, CB // 16, grp_body, 0)

        def scatter(b):
            return pltpu.async_copy(sc_b[b], acc.at[cb_b[b]], sem_s[b],
                                    add=True)

        def wait_g(b):
            pltpu.make_async_copy(g_hbm.at[gidx_b[b]], rows_b[b],
                                  sem_g[b]).wait()

        def wait_s(b):
            pltpu.make_async_copy(sc_b[b], acc.at[cb_b[b]],
                                  sem_s[b]).wait()

        meta_load(0, 0)
        meta_wait(0, 0)
        stage_g(0)
        gather(0)
        meta_load(1, 1)

        def pair_body(kk, _):
            a = 2 * kk
            meta_wait(a + 1, 1)
            stage_g(1)
            gather(1)
            wait_g(0)

            @pl.when(kk > 0)
            def _():
                wait_s(0)
            scale(0)
            stage_c(0)
            scatter(0)
            meta_load(a + 2, 0)
            meta_wait(a + 2, 0)
            stage_g(0)
            gather(0)
            wait_g(1)

            @pl.when(kk > 0)
            def _():
                wait_s(1)
            scale(1)
            stage_c(1)
            scatter(1)
            meta_load(jnp.minimum(a + 3, NCB - 1), 1)
            return 0
        lax.fori_loop(0, (NCB - 1) // 2, pair_body, 0)

        wait_g(0)
        wait_s(0)
        scale(0)
        stage_c(0)
        scatter(0)
        meta_wait(NCB - 1, 1)
        wait_s(0)
        wait_s(1)
        plsc.subcore_barrier()

        def wr_body(t, _):
            g = t * NS + s

            @pl.when(g < NGT)
            def _():
                pltpu.sync_copy(acc.at[pl.ds(g * 16, 16)],
                                out_hbm.at[pl.ds(f * N + g * 16, 16)])
            return 0
        lax.fori_loop(0, NGL, wr_body, 0)
        plsc.subcore_barrier()


@functools.partial(
    pl.kernel,
    out_type=jax.ShapeDtypeStruct((E,), f32),
    mesh=_mesh,
    compiler_params=pltpu.CompilerParams(needs_layout_passes=False),
    scratch_types=[
        pltpu.VMEM((EPT_G,), i32),
        pltpu.VMEM((EPT_G,), i32),
        pltpu.VMEM((CS, D_H), f32),
        pltpu.VMEM((CS, D_H), f32),
        pltpu.VMEM((CS, D_H), f32),
        pltpu.VMEM((CS, D_H), f32),
        pltpu.VMEM((48,), f32),
        pltpu.SemaphoreType.DMA,
        pltpu.SemaphoreType.DMA,
        pltpu.SemaphoreType.DMA,
        pltpu.SemaphoreType.DMA,
    ],
)
def _score_kernel(h_hbm, row_hbm, col_hbm, out_hbm,
                  rall, call, hr0, hr1, hc0, hc1, sv,
                  semr0, semr1, semc0, semc1):
    c = lax.axis_index("c")
    s = lax.axis_index("s")
    wid = s * NC + c
    ebase = wid * EPT_G
    iota16 = lax.iota(i32, 16)

    pltpu.sync_copy(row_hbm.at[pl.ds(ebase, EPT_G)], rall)
    pltpu.sync_copy(col_hbm.at[pl.ds(ebase, EPT_G)], call)

    hr_b = (hr0, hr1)
    hc_b = (hc0, hc1)
    semr = (semr0, semr1)
    semc = (semc0, semc1)

    def gather(k, b):
        idx_r = rall.at[pl.ds(k * CS, CS)]
        idx_c = call.at[pl.ds(k * CS, CS)]
        pltpu.async_copy(h_hbm.at[idx_r], hr_b[b], semr[b])
        pltpu.async_copy(h_hbm.at[idx_c], hc_b[b], semc[b])

    def wait(k, b):
        idx_r = rall.at[pl.ds(k * CS, CS)]
        idx_c = call.at[pl.ds(k * CS, CS)]
        pltpu.make_async_copy(h_hbm.at[idx_r], hr_b[b], semr[b]).wait()
        pltpu.make_async_copy(h_hbm.at[idx_c], hc_b[b], semc[b]).wait()

    def compute(k, b):
        hr = hr_b[b]
        hc = hc_b[b]
        for gi in range((CS + 15) // 16):
            nv = min(16, CS - gi * 16)

            def dot_body(i, outv):
                e = gi * 16 + i
                accv = hr[e, pl.ds(0, 16)] * hc[e, pl.ds(0, 16)]
                for u in range(1, D_H // 16):
                    sl = pl.ds(u * 16, 16)
                    accv = accv + hr[e, sl] * hc[e, sl]
                d = jnp.sum(accv)
                return jnp.where(iota16 == i, d, outv)
            outv = lax.fori_loop(0, nv, dot_body, jnp.zeros((16,), f32))
            sv[pl.ds(gi * 16, 16)] = 1.0 / (1.0 + jnp.exp(-outv))
        pltpu.sync_copy(sv.at[pl.ds(0, CS)],
                        out_hbm.at[pl.ds(ebase + k * CS, CS)])

    gather(0, 0)

    def pair_body(kk, _):
        a = 2 * kk
        gather(a + 1, 1)
        wait(a, 0)
        compute(a, 0)
        gather(a + 2, 0)
        wait(a + 1, 1)
        compute(a + 1, 1)
        return 0
    lax.fori_loop(0, (NCS - 1) // 2, pair_body, 0)

    wait(NCS - 1, 0)
    compute(NCS - 1, 0)


def _k1_body(x_ref, w1_ref, d0_ref, d1_ref, out_ref):
    deg = 1.0 + d0_ref[...] + d1_ref[...]
    dinv = lax.rsqrt(deg)
    h = jnp.dot(x_ref[...], w1_ref[...], preferred_element_type=f32)
    out_ref[...] = h * dinv


def _k1(x, W1, d0, d1):
    return pl.pallas_call(
        _k1_body,
        grid=(NRB, NFC),
        in_specs=[
            pl.BlockSpec((RB, D_IN), lambda i, f: (i, 0)),
            pl.BlockSpec((D_IN, FC), lambda i, f: (0, f)),
            pl.BlockSpec((RB, 1), lambda i, f: (i, 0)),
            pl.BlockSpec((RB, 1), lambda i, f: (i, 0)),
        ],
        out_specs=pl.BlockSpec((RB, FC), lambda i, f: (f * NRB + i, 0)),
        out_shape=jax.ShapeDtypeStruct((NFC * N, FC), f32),
    )(x, W1, d0, d1)


def _k2_body(s10, s11, s12, s13, g10, g11, g12, g13, d0_ref, d1_ref,
             w2_ref, b1_ref, out_ref):
    deg = 1.0 + d0_ref[...] + d1_ref[...]
    dinv = lax.rsqrt(deg)
    s_k = (s10, s11, s12, s13)
    g_k = (g10, g11, g12, g13)
    z = jnp.concatenate(
        [jnp.maximum(dinv * (s_k[k][...] + g_k[k][...])
                     + b1_ref[k, 0][None, :], 0.0)
         for k in range(NFC)], axis=1)
    out_ref[...] = jnp.dot(z, w2_ref[...], preferred_element_type=f32) * dinv


def _k2(s1, g1, d0, d1, W2, b1r):
    def chunk_spec(k):
        return pl.BlockSpec((RB, FC), lambda i, f, k=k: (k * NRB + i, 0))

    return pl.pallas_call(
        _k2_body,
        grid=(NRB, NFC),
        in_specs=(
            [chunk_spec(k) for k in range(NFC)] * 2
            + [
                pl.BlockSpec((RB, 1), lambda i, f: (i, 0)),
                pl.BlockSpec((RB, 1), lambda i, f: (i, 0)),
                pl.BlockSpec((D_H, FC), lambda i, f: (0, f)),
                pl.BlockSpec((NFC, 1, FC), lambda i, f: (0, 0, 0)),
            ]
        ),
        out_specs=pl.BlockSpec((RB, FC), lambda i, f: (f * NRB + i, 0)),
        out_shape=jax.ShapeDtypeStruct((NFC * N, FC), f32),
        compiler_params=pltpu.CompilerParams(
            dimension_semantics=("parallel", "parallel")),
    )(s1, s1, s1, s1, g1, g1, g1, g1, d0, d1, W2, b1r)


def _k3_body(s2_ref, g2_ref, d0_ref, d1_ref, b2_ref, out_ref):
    deg = 1.0 + d0_ref[...] + d1_ref[...]
    dinv = lax.rsqrt(deg)
    out_ref[...] = dinv * (s2_ref[...] + g2_ref[...]) + b2_ref[0]


def _k3(s2, g2, d0, d1, b2r):
    return pl.pallas_call(
        _k3_body,
        grid=(NRB, NFC),
        in_specs=[
            pl.BlockSpec((RB, FC), lambda i, f: (f * NRB + i, 0)),
            pl.BlockSpec((RB, FC), lambda i, f: (f * NRB + i, 0)),
            pl.BlockSpec((RB, 1), lambda i, f: (i, 0)),
            pl.BlockSpec((RB, 1), lambda i, f: (i, 0)),
            pl.BlockSpec((1, 1, FC), lambda i, f: (f, 0, 0)),
        ],
        out_specs=pl.BlockSpec((RB, FC), lambda i, f: (i, f)),
        out_shape=jax.ShapeDtypeStruct((N, D_H), f32),
    )(s2, g2, d0, d1, b2r)


def kernel(x, edge_index, edge_weight, W1, b1, W2, b2):
    row = edge_index[0].astype(i32)
    col = edge_index[1].astype(i32)
    ew = edge_weight.astype(f32)
    b1r = b1.reshape(NFC, 1, FC)
    b2r = b2.reshape(NFC, 1, FC)

    deg_part = _deg_kernel(col, ew)
    d0 = deg_part[0, :N].reshape(N, 1)
    d1 = deg_part[1, :N].reshape(N, 1)
    g1 = _k1(x, W1, d0, d1)
    s1 = _prop_kernel(g1, row, col, ew)
    g2 = _k2(s1, g1, d0, d1, W2, b1r)
    s2 = _prop_kernel(g2, row, col, ew)
    h = _k3(s2, g2, d0, d1, b2r)
    return _score_kernel(h, row, col)


# speedup vs baseline: 8.4327x; 1.1457x over previous
import functools

import jax
import jax.numpy as jnp
from jax import lax
from jax.experimental import pallas as pl
from jax.experimental.pallas import tpu as pltpu
from jax.experimental.pallas import tpu_sc as plsc

N = 10000
E = 160000
D_IN = 256
D_H = 512
NC = 2
NS = 16
NW = NC * NS
FC = 128
NFC = D_H // FC
RB = 400
NRB = N // RB

EPT_G = E // NW
EPT_C = E // NS
CB = 80
NCB = EPT_C // CB
CS = 40
NCS = EPT_G // CS
RPT = N // NS

_mesh = plsc.VectorSubcoreMesh(core_axis_name="c", subcore_axis_name="s")

f32 = jnp.float32
i32 = jnp.int32


N_PAD = 10240
SLC = N_PAD // NS


@functools.partial(
    pl.kernel,
    out_type=jax.ShapeDtypeStruct((NC, N_PAD), f32),
    mesh=_mesh,
    compiler_params=pltpu.CompilerParams(needs_layout_passes=False),
    scratch_types=[
        pltpu.VMEM((N_PAD,), f32),
        pltpu.VMEM_SHARED((NS, N_PAD), f32),
        pltpu.VMEM((EPT_G + 16,), i32),
        pltpu.VMEM((EPT_G + 16,), f32),
        pltpu.VMEM((SLC,), f32),
        pltpu.VMEM((SLC,), f32),
    ],
)
def _deg_kernel(col_hbm, ew_hbm, out_hbm, acc, slots, colbuf, ewbuf,
                tmp, sumb):
    c = lax.axis_index("c")
    s = lax.axis_index("s")

    def zero_body(i, _):
        acc[pl.ds(i * 16, 16)] = jnp.zeros((16,), f32)
        return 0
    lax.fori_loop(0, N_PAD // 16, zero_body, 0)

    base = c * (E // NC) + s * EPT_G
    pltpu.sync_copy(col_hbm.at[pl.ds(base, EPT_G)], colbuf.at[pl.ds(0, EPT_G)])
    pltpu.sync_copy(ew_hbm.at[pl.ds(base, EPT_G)], ewbuf.at[pl.ds(0, EPT_G)])

    iota = lax.iota(i32, 16)
    ngroups = (EPT_G + 15) // 16

    def grp_body(g, _):
        off = g * 16
        m = (off + iota) < EPT_G
        cv = colbuf[pl.ds(off, 16)]
        wv = ewbuf[pl.ds(off, 16)]
        plsc.addupdate_scatter(acc, [cv], wv, mask=m)
        return 0
    lax.fori_loop(0, ngroups, grp_body, 0)

    pltpu.sync_copy(acc, slots.at[s])
    plsc.subcore_barrier()

    for p in range(NS):
        pltpu.sync_copy(slots.at[p, pl.ds(s * SLC, SLC)], tmp)
        for u in range(SLC // 16):
            sl = pl.ds(u * 16, 16)
            if p == 0:
                sumb[sl] = tmp[sl]
            else:
                sumb[sl] = sumb[sl] + tmp[sl]
    pltpu.sync_copy(sumb, out_hbm.at[c, pl.ds(s * SLC, SLC)])


@functools.partial(
    pl.kernel,
    out_type=jax.ShapeDtypeStruct((NFC * N, FC), f32),
    mesh=_mesh,
    compiler_params=pltpu.CompilerParams(needs_layout_passes=False),
    scratch_types=[
        pltpu.VMEM_SHARED((N, FC), f32),
        pltpu.VMEM((CB, FC), f32),
        pltpu.VMEM((CB, FC), f32),
        pltpu.VMEM((CB, FC), f32),
        pltpu.VMEM((CB, FC), f32),
        pltpu.VMEM((CB,), i32),
        pltpu.VMEM((CB,), i32),
        pltpu.VMEM((CB,), i32),
        pltpu.VMEM((CB,), i32),
        pltpu.VMEM((CB,), i32),
        pltpu.VMEM((CB,), i32),
        pltpu.VMEM((CB,), i32),
        pltpu.VMEM((CB,), i32),
        pltpu.VMEM((CB,), f32),
        pltpu.VMEM((CB,), f32),
        pltpu.VMEM((16, FC), f32),
        pltpu.SemaphoreType.DMA,
        pltpu.SemaphoreType.DMA,
        pltpu.SemaphoreType.DMA,
        pltpu.SemaphoreType.DMA,
        pltpu.SemaphoreType.DMA,
        pltpu.SemaphoreType.DMA,
    ],
)
def _prop_kernel(g_hbm, row_hbm, col_hbm, ew_hbm, out_hbm,
                 acc, rows0, rows1, sc0, sc1, gidx0, gidx1, cb0, cb1,
                 rw0, rw1, cl0, cl1, ew0, ew1, zbuf,
                 sem_g0, sem_g1, sem_s0, sem_s1, sem_m0, sem_m1):
    c = lax.axis_index("c")
    s = lax.axis_index("s")
    NGT = N // 16
    NGL = (NGT + NS - 1) // NS
    iota16 = lax.iota(i32, 16)

    def zb_body(r, _):
        for u in range(FC // 16):
            zbuf[r, pl.ds(u * 16, 16)] = jnp.zeros((16,), f32)
        return 0
    lax.fori_loop(0, 16, zb_body, 0)

    ebase = s * EPT_C
    rows_b = (rows0, rows1)
    sc_b = (sc0, sc1)
    gidx_b = (gidx0, gidx1)
    cb_b = (cb0, cb1)
    rw_b = (rw0, rw1)
    cl_b = (cl0, cl1)
    ew_b = (ew0, ew1)
    sem_g = (sem_g0, sem_g1)
    sem_s = (sem_s0, sem_s1)
    sem_m = (sem_m0, sem_m1)

    def meta_load(k, b):
        src = pl.ds(ebase + k * CB, CB)
        pltpu.async_copy(row_hbm.at[src], rw_b[b], sem_m[b])
        pltpu.async_copy(col_hbm.at[src], cl_b[b], sem_m[b])
        pltpu.async_copy(ew_hbm.at[src], ew_b[b], sem_m[b])

    def meta_wait(k, b):
        src = pl.ds(ebase + k * CB, CB)
        pltpu.make_async_copy(row_hbm.at[src], rw_b[b], sem_m[b]).wait()
        pltpu.make_async_copy(col_hbm.at[src], cl_b[b], sem_m[b]).wait()
        pltpu.make_async_copy(ew_hbm.at[src], ew_b[b], sem_m[b]).wait()

    for j in range(NFC // NC):
        f = c * (NFC // NC) + j
        foff = f * N

        def zero_body(t, _):
            g = t * NS + s

            @pl.when(g < NGT)
            def _():
                pltpu.sync_copy(zbuf, acc.at[pl.ds(g * 16, 16)])
            return 0
        lax.fori_loop(0, NGL, zero_body, 0)
        plsc.subcore_barrier()

        def stage_g(b):
            for g in range(CB // 16):
                sl = pl.ds(g * 16, 16)
                gidx_b[b][sl] = rw_b[b][sl] + foff

        def stage_c(b):
            for g in range(CB // 16):
                sl = pl.ds(g * 16, 16)
                cb_b[b][sl] = cl_b[b][sl]

        def gather(b):
            return pltpu.async_copy(g_hbm.at[gidx_b[b]], rows_b[b],
                                    sem_g[b])

        def scale(b):
            zeros16 = jnp.zeros((16,), i32)

            def grp_body(gi, _):
                for l in range(16):
                    e = gi * 16 + l
                    wv = plsc.load_gather(ew_b[b], [zeros16 + e])
                    for u in range(FC // 16):
                        sl = pl.ds(u * 16, 16)
                        sc_b[b][e, sl] = rows_b[b][e, sl] * wv
                return 0
            lax.fori_loop(0, CB // 16, grp_body, 0)

        def scatter(b):
            return pltpu.async_copy(sc_b[b], acc.at[cb_b[b]], sem_s[b],
                                    add=True)

        def wait_g(b):
            pltpu.make_async_copy(g_hbm.at[gidx_b[b]], rows_b[b],
                                  sem_g[b]).wait()

        def wait_s(b):
            pltpu.make_async_copy(sc_b[b], acc.at[cb_b[b]],
                                  sem_s[b]).wait()

        meta_load(0, 0)
        meta_wait(0, 0)
        stage_g(0)
        gather(0)
        meta_load(1, 1)

        def pair_body(kk, _):
            a = 2 * kk
            meta_wait(a + 1, 1)
            stage_g(1)
            gather(1)
            wait_g(0)

            @pl.when(kk > 0)
            def _():
                wait_s(0)
            scale(0)
            stage_c(0)
            scatter(0)
            meta_load(a + 2, 0)
            meta_wait(a + 2, 0)
            stage_g(0)
            gather(0)
            wait_g(1)

            @pl.when(kk > 0)
            def _():
                wait_s(1)
            scale(1)
            stage_c(1)
            scatter(1)
            meta_load(jnp.minimum(a + 3, NCB - 1), 1)
            return 0
        lax.fori_loop(0, (NCB - 1) // 2, pair_body, 0)

        wait_g(0)
        wait_s(0)
        scale(0)
        stage_c(0)
        scatter(0)
        meta_wait(NCB - 1, 1)
        wait_s(0)
        wait_s(1)
        plsc.subcore_barrier()

        def wr_body(t, _):
            g = t * NS + s

            @pl.when(g < NGT)
            def _():
                pltpu.sync_copy(acc.at[pl.ds(g * 16, 16)],
                                out_hbm.at[pl.ds(f * N + g * 16, 16)])
            return 0
        lax.fori_loop(0, NGL, wr_body, 0)
        plsc.subcore_barrier()


@functools.partial(
    pl.kernel,
    out_type=jax.ShapeDtypeStruct((E,), f32),
    mesh=_mesh,
    compiler_params=pltpu.CompilerParams(needs_layout_passes=False),
    scratch_types=[
        pltpu.VMEM((EPT_G,), i32),
        pltpu.VMEM((EPT_G,), i32),
        pltpu.VMEM((CS, D_H // 2), i32),
        pltpu.VMEM((CS, D_H // 2), i32),
        pltpu.VMEM((CS, D_H // 2), i32),
        pltpu.VMEM((CS, D_H // 2), i32),
        pltpu.VMEM((48,), f32),
        pltpu.SemaphoreType.DMA,
        pltpu.SemaphoreType.DMA,
        pltpu.SemaphoreType.DMA,
        pltpu.SemaphoreType.DMA,
    ],
)
def _score_kernel(h_hbm, row_hbm, col_hbm, out_hbm,
                  rall, call, hr0, hr1, hc0, hc1, sv,
                  semr0, semr1, semc0, semc1):
    c = lax.axis_index("c")
    s = lax.axis_index("s")
    wid = s * NC + c
    ebase = wid * EPT_G
    iota16 = lax.iota(i32, 16)

    pltpu.sync_copy(row_hbm.at[pl.ds(ebase, EPT_G)], rall)
    pltpu.sync_copy(col_hbm.at[pl.ds(ebase, EPT_G)], call)

    hr_b = (hr0, hr1)
    hc_b = (hc0, hc1)
    semr = (semr0, semr1)
    semc = (semc0, semc1)

    def gather(k, b):
        idx_r = rall.at[pl.ds(k * CS, CS)]
        idx_c = call.at[pl.ds(k * CS, CS)]
        pltpu.async_copy(h_hbm.at[idx_r], hr_b[b], semr[b])
        pltpu.async_copy(h_hbm.at[idx_c], hc_b[b], semc[b])

    def wait(k, b):
        idx_r = rall.at[pl.ds(k * CS, CS)]
        idx_c = call.at[pl.ds(k * CS, CS)]
        pltpu.make_async_copy(h_hbm.at[idx_r], hr_b[b], semr[b]).wait()
        pltpu.make_async_copy(h_hbm.at[idx_c], hc_b[b], semc[b]).wait()

    def compute(k, b):
        hr = hr_b[b]
        hc = hc_b[b]
        for gi in range((CS + 15) // 16):
            nv = min(16, CS - gi * 16)

            def dot_body(i, outv):
                e = gi * 16 + i
                accv = jnp.zeros((16,), f32)
                for u in range(D_H // 32):
                    sl = pl.ds(u * 16, 16)
                    ar = plsc.bitcast(hr[e, sl], jnp.bfloat16)
                    ac = plsc.bitcast(hc[e, sl], jnp.bfloat16)
                    p0, p1 = plsc.unpack(
                        ar * ac, format=plsc.PackFormat.INTERLEAVED)
                    accv = accv + p0 + p1
                d = jnp.sum(accv)
                return jnp.where(iota16 == i, d, outv)
            outv = lax.fori_loop(0, nv, dot_body, jnp.zeros((16,), f32))
            sv[pl.ds(gi * 16, 16)] = 1.0 / (1.0 + jnp.exp(-outv))
        pltpu.sync_copy(sv.at[pl.ds(0, CS)],
                        out_hbm.at[pl.ds(ebase + k * CS, CS)])

    gather(0, 0)

    def pair_body(kk, _):
        a = 2 * kk
        gather(a + 1, 1)
        wait(a, 0)
        compute(a, 0)
        gather(a + 2, 0)
        wait(a + 1, 1)
        compute(a + 1, 1)
        return 0
    lax.fori_loop(0, (NCS - 1) // 2, pair_body, 0)

    wait(NCS - 1, 0)
    compute(NCS - 1, 0)


def _k1_body(x_ref, w1_ref, d0_ref, d1_ref, out_ref):
    deg = 1.0 + d0_ref[...] + d1_ref[...]
    dinv = lax.rsqrt(deg)
    h = jnp.dot(x_ref[...], w1_ref[...], preferred_element_type=f32)
    out_ref[...] = h * dinv


def _k1(x, W1, d0, d1):
    return pl.pallas_call(
        _k1_body,
        grid=(NRB, NFC),
        in_specs=[
            pl.BlockSpec((RB, D_IN), lambda i, f: (i, 0)),
            pl.BlockSpec((D_IN, FC), lambda i, f: (0, f)),
            pl.BlockSpec((RB, 1), lambda i, f: (i, 0)),
            pl.BlockSpec((RB, 1), lambda i, f: (i, 0)),
        ],
        out_specs=pl.BlockSpec((RB, FC), lambda i, f: (f * NRB + i, 0)),
        out_shape=jax.ShapeDtypeStruct((NFC * N, FC), f32),
    )(x, W1, d0, d1)


def _k2_body(s10, s11, s12, s13, g10, g11, g12, g13, d0_ref, d1_ref,
             w2_ref, b1_ref, out_ref):
    deg = 1.0 + d0_ref[...] + d1_ref[...]
    dinv = lax.rsqrt(deg)
    s_k = (s10, s11, s12, s13)
    g_k = (g10, g11, g12, g13)
    z = jnp.concatenate(
        [jnp.maximum(dinv * (s_k[k][...] + g_k[k][...])
                     + b1_ref[k, 0][None, :], 0.0)
         for k in range(NFC)], axis=1)
    out_ref[...] = jnp.dot(z, w2_ref[...], preferred_element_type=f32) * dinv


def _k2(s1, g1, d0, d1, W2, b1r):
    def chunk_spec(k):
        return pl.BlockSpec((RB, FC), lambda i, f, k=k: (k * NRB + i, 0))

    return pl.pallas_call(
        _k2_body,
        grid=(NRB, NFC),
        in_specs=(
            [chunk_spec(k) for k in range(NFC)] * 2
            + [
                pl.BlockSpec((RB, 1), lambda i, f: (i, 0)),
                pl.BlockSpec((RB, 1), lambda i, f: (i, 0)),
                pl.BlockSpec((D_H, FC), lambda i, f: (0, f)),
                pl.BlockSpec((NFC, 1, FC), lambda i, f: (0, 0, 0)),
            ]
        ),
        out_specs=pl.BlockSpec((RB, FC), lambda i, f: (f * NRB + i, 0)),
        out_shape=jax.ShapeDtypeStruct((NFC * N, FC), f32),
        compiler_params=pltpu.CompilerParams(
            dimension_semantics=("parallel", "parallel")),
    )(s1, s1, s1, s1, g1, g1, g1, g1, d0, d1, W2, b1r)


def _k3_body(s20, s21, s22, s23, g20, g21, g22, g23, d0_ref, d1_ref,
             b2_ref, out_ref):
    deg = 1.0 + d0_ref[...] + d1_ref[...]
    dinv = lax.rsqrt(deg)
    s_k = (s20, s21, s22, s23)
    g_k = (g20, g21, g22, g23)
    parts = [(dinv * (s_k[k][...] + g_k[k][...]) + b2_ref[k, 0][None, :]
              ).astype(jnp.bfloat16)
             for k in range(NFC)]
    ha = jnp.concatenate(parts[:2], axis=1)
    hb = jnp.concatenate(parts[2:], axis=1)
    wa = lax.bitcast_convert_type(ha, jnp.uint16).astype(i32)
    wb = lax.bitcast_convert_type(hb, jnp.uint16).astype(i32)
    out_ref[...] = wa | (wb << 16)


def _k3(s2, g2, d0, d1, b2r):
    def chunk_spec(k):
        return pl.BlockSpec((RB, FC), lambda i, k=k: (k * NRB + i, 0))

    return pl.pallas_call(
        _k3_body,
        grid=(NRB,),
        in_specs=(
            [chunk_spec(k) for k in range(NFC)] * 2
            + [
                pl.BlockSpec((RB, 1), lambda i: (i, 0)),
                pl.BlockSpec((RB, 1), lambda i: (i, 0)),
                pl.BlockSpec((NFC, 1, FC), lambda i: (0, 0, 0)),
            ]
        ),
        out_specs=pl.BlockSpec((RB, D_H // 2), lambda i: (i, 0)),
        out_shape=jax.ShapeDtypeStruct((N, D_H // 2), i32),
    )(s2, s2, s2, s2, g2, g2, g2, g2, d0, d1, b2r)


def kernel(x, edge_index, edge_weight, W1, b1, W2, b2):
    row = edge_index[0].astype(i32)
    col = edge_index[1].astype(i32)
    ew = edge_weight.astype(f32)
    b1r = b1.reshape(NFC, 1, FC)
    b2r = b2.reshape(NFC, 1, FC)

    deg_part = _deg_kernel(col, ew)
    d0 = deg_part[0, :N].reshape(N, 1)
    d1 = deg_part[1, :N].reshape(N, 1)
    g1 = _k1(x, W1, d0, d1)
    s1 = _prop_kernel(g1, row, col, ew)
    g2 = _k2(s1, g1, d0, d1, W2, b1r)
    s2 = _prop_kernel(g2, row, col, ew)
    h = _k3(s2, g2, d0, d1, b2r)
    return _score_kernel(h, row, col)


# speedup vs baseline: 8.4821x; 1.0059x over previous
import functools

import jax
import jax.numpy as jnp
from jax import lax
from jax.experimental import pallas as pl
from jax.experimental.pallas import tpu as pltpu
from jax.experimental.pallas import tpu_sc as plsc

N = 10000
E = 160000
D_IN = 256
D_H = 512
NC = 2
NS = 16
NW = NC * NS
FC = 128
NFC = D_H // FC
RB = 400
NRB = N // RB

EPT_G = E // NW
EPT_C = E // NS
CB = 80
NCB = EPT_C // CB
CS = 40
NCS = EPT_G // CS
RPT = N // NS

_mesh = plsc.VectorSubcoreMesh(core_axis_name="c", subcore_axis_name="s")

f32 = jnp.float32
i32 = jnp.int32


N_PAD = 10240
SLC = N_PAD // NS


@functools.partial(
    pl.kernel,
    out_type=jax.ShapeDtypeStruct((NC, N_PAD), f32),
    mesh=_mesh,
    compiler_params=pltpu.CompilerParams(needs_layout_passes=False),
    scratch_types=[
        pltpu.VMEM((N_PAD,), f32),
        pltpu.VMEM_SHARED((NS, N_PAD), f32),
        pltpu.VMEM((EPT_G + 16,), i32),
        pltpu.VMEM((EPT_G + 16,), f32),
        pltpu.VMEM((SLC,), f32),
        pltpu.VMEM((SLC,), f32),
    ],
)
def _deg_kernel(col_hbm, ew_hbm, out_hbm, acc, slots, colbuf, ewbuf,
                tmp, sumb):
    c = lax.axis_index("c")
    s = lax.axis_index("s")

    def zero_body(i, _):
        acc[pl.ds(i * 16, 16)] = jnp.zeros((16,), f32)
        return 0
    lax.fori_loop(0, N_PAD // 16, zero_body, 0)

    base = c * (E // NC) + s * EPT_G
    pltpu.sync_copy(col_hbm.at[pl.ds(base, EPT_G)], colbuf.at[pl.ds(0, EPT_G)])
    pltpu.sync_copy(ew_hbm.at[pl.ds(base, EPT_G)], ewbuf.at[pl.ds(0, EPT_G)])

    iota = lax.iota(i32, 16)
    ngroups = (EPT_G + 15) // 16

    def grp_body(g, _):
        off = g * 16
        m = (off + iota) < EPT_G
        cv = colbuf[pl.ds(off, 16)]
        wv = ewbuf[pl.ds(off, 16)]
        plsc.addupdate_scatter(acc, [cv], wv, mask=m)
        return 0
    lax.fori_loop(0, ngroups, grp_body, 0)

    pltpu.sync_copy(acc, slots.at[s])
    plsc.subcore_barrier()

    for p in range(NS):
        pltpu.sync_copy(slots.at[p, pl.ds(s * SLC, SLC)], tmp)
        for u in range(SLC // 16):
            sl = pl.ds(u * 16, 16)
            if p == 0:
                sumb[sl] = tmp[sl]
            else:
                sumb[sl] = sumb[sl] + tmp[sl]
    pltpu.sync_copy(sumb, out_hbm.at[c, pl.ds(s * SLC, SLC)])


@functools.partial(
    pl.kernel,
    out_type=jax.ShapeDtypeStruct((NFC * N, FC), f32),
    mesh=_mesh,
    compiler_params=pltpu.CompilerParams(needs_layout_passes=False),
    scratch_types=[
        pltpu.VMEM_SHARED((N, FC), f32),
        pltpu.VMEM((CB, FC), f32),
        pltpu.VMEM((CB, FC), f32),
        pltpu.VMEM((CB, FC), f32),
        pltpu.VMEM((CB, FC), f32),
        pltpu.VMEM((CB,), i32),
        pltpu.VMEM((CB,), i32),
        pltpu.VMEM((CB,), i32),
        pltpu.VMEM((CB,), i32),
        pltpu.VMEM((CB,), i32),
        pltpu.VMEM((CB,), i32),
        pltpu.VMEM((CB,), i32),
        pltpu.VMEM((CB,), i32),
        pltpu.VMEM((CB,), f32),
        pltpu.VMEM((CB,), f32),
        pltpu.VMEM((16, FC), f32),
        pltpu.SemaphoreType.DMA,
        pltpu.SemaphoreType.DMA,
        pltpu.SemaphoreType.DMA,
        pltpu.SemaphoreType.DMA,
        pltpu.SemaphoreType.DMA,
        pltpu.SemaphoreType.DMA,
    ],
)
def _prop_kernel(g_hbm, row_hbm, col_hbm, ew_hbm, out_hbm,
                 acc, rows0, rows1, sc0, sc1, gidx0, gidx1, cb0, cb1,
                 rw0, rw1, cl0, cl1, ew0, ew1, zbuf,
                 sem_g0, sem_g1, sem_s0, sem_s1, sem_m0, sem_m1):
    c = lax.axis_index("c")
    s = lax.axis_index("s")
    NGT = N // 16
    NGL = (NGT + NS - 1) // NS
    iota16 = lax.iota(i32, 16)

    def zb_body(r, _):
        for u in range(FC // 16):
            zbuf[r, pl.ds(u * 16, 16)] = jnp.zeros((16,), f32)
        return 0
    lax.fori_loop(0, 16, zb_body, 0)

    ebase = s * EPT_C
    rows_b = (rows0, rows1)
    sc_b = (sc0, sc1)
    gidx_b = (gidx0, gidx1)
    cb_b = (cb0, cb1)
    rw_b = (rw0, rw1)
    cl_b = (cl0, cl1)
    ew_b = (ew0, ew1)
    sem_g = (sem_g0, sem_g1)
    sem_s = (sem_s0, sem_s1)
    sem_m = (sem_m0, sem_m1)

    def meta_load(k, b):
        src = pl.ds(ebase + k * CB, CB)
        pltpu.async_copy(row_hbm.at[src], rw_b[b], sem_m[b])
        pltpu.async_copy(col_hbm.at[src], cl_b[b], sem_m[b])
        pltpu.async_copy(ew_hbm.at[src], ew_b[b], sem_m[b])

    def meta_wait(k, b):
        src = pl.ds(ebase + k * CB, CB)
        pltpu.make_async_copy(row_hbm.at[src], rw_b[b], sem_m[b]).wait()
        pltpu.make_async_copy(col_hbm.at[src], cl_b[b], sem_m[b]).wait()
        pltpu.make_async_copy(ew_hbm.at[src], ew_b[b], sem_m[b]).wait()

    for j in range(NFC // NC):
        f = c * (NFC // NC) + j
        foff = f * N

        def zero_body(t, _):
            g = t * NS + s

            @pl.when(g < NGT)
            def _():
                pltpu.sync_copy(zbuf, acc.at[pl.ds(g * 16, 16)])
            return 0
        lax.fori_loop(0, NGL, zero_body, 0)
        plsc.subcore_barrier()

        def stage_g(b):
            for g in range(CB // 16):
                sl = pl.ds(g * 16, 16)
                gidx_b[b][sl] = rw_b[b][sl] + foff

        def stage_c(b):
            for g in range(CB // 16):
                sl = pl.ds(g * 16, 16)
                cb_b[b][sl] = cl_b[b][sl]

        def gather(b):
            return pltpu.async_copy(g_hbm.at[gidx_b[b]], rows_b[b],
                                    sem_g[b])

        def scale(b):
            zeros16 = jnp.zeros((16,), i32)

            def grp_body(gi, _):
                for l in range(16):
                    e = gi * 16 + l
                    wv = plsc.load_gather(ew_b[b], [zeros16 + e])
                    for u in range(FC // 16):
                        sl = pl.ds(u * 16, 16)
                        sc_b[b][e, sl] = rows_b[b][e, sl] * wv
                return 0
            lax.fori_loop(0, CB // 16, grp_body, 0)

        def scatter(b):
            return pltpu.async_copy(sc_b[b], acc.at[cb_b[b]], sem_s[b],
                                    add=True)

        def wait_g(b):
            pltpu.make_async_copy(g_hbm.at[gidx_b[b]], rows_b[b],
                                  sem_g[b]).wait()

        def wait_s(b):
            pltpu.make_async_copy(sc_b[b], acc.at[cb_b[b]],
                                  sem_s[b]).wait()

        meta_load(0, 0)
        meta_wait(0, 0)
        stage_g(0)
        gather(0)
        meta_load(1, 1)

        def pair_body(kk, _):
            a = 2 * kk
            meta_wait(a + 1, 1)
            stage_g(1)
            gather(1)
            wait_g(0)

            @pl.when(kk > 0)
            def _():
                wait_s(0)
            scale(0)
            stage_c(0)
            scatter(0)
            meta_load(a + 2, 0)
            meta_wait(a + 2, 0)
            stage_g(0)
            gather(0)
            wait_g(1)

            @pl.when(kk > 0)
            def _():
                wait_s(1)
            scale(1)
            stage_c(1)
            scatter(1)
            meta_load(jnp.minimum(a + 3, NCB - 1), 1)
            return 0
        lax.fori_loop(0, (NCB - 1) // 2, pair_body, 0)

        wait_g(0)
        wait_s(0)
        scale(0)
        stage_c(0)
        scatter(0)
        meta_wait(NCB - 1, 1)
        wait_s(0)
        wait_s(1)
        plsc.subcore_barrier()

        def wr_body(t, _):
            g = t * NS + s

            @pl.when(g < NGT)
            def _():
                pltpu.sync_copy(acc.at[pl.ds(g * 16, 16)],
                                out_hbm.at[pl.ds(f * N + g * 16, 16)])
            return 0
        lax.fori_loop(0, NGL, wr_body, 0)
        plsc.subcore_barrier()


@functools.partial(
    pl.kernel,
    out_type=jax.ShapeDtypeStruct((E,), f32),
    mesh=_mesh,
    compiler_params=pltpu.CompilerParams(needs_layout_passes=False),
    scratch_types=[
        pltpu.VMEM((EPT_G,), i32),
        pltpu.VMEM((EPT_G,), i32),
        pltpu.VMEM((CS, D_H // 2), i32),
        pltpu.VMEM((CS, D_H // 2), i32),
        pltpu.VMEM((CS, D_H // 2), i32),
        pltpu.VMEM((CS, D_H // 2), i32),
        pltpu.VMEM((48,), f32),
        pltpu.SemaphoreType.DMA,
        pltpu.SemaphoreType.DMA,
        pltpu.SemaphoreType.DMA,
        pltpu.SemaphoreType.DMA,
    ],
)
def _score_kernel(h_hbm, row_hbm, col_hbm, out_hbm,
                  rall, call, hr0, hr1, hc0, hc1, sv,
                  semr0, semr1, semc0, semc1):
    c = lax.axis_index("c")
    s = lax.axis_index("s")
    wid = s * NC + c
    ebase = wid * EPT_G
    iota16 = lax.iota(i32, 16)

    pltpu.sync_copy(row_hbm.at[pl.ds(ebase, EPT_G)], rall)
    pltpu.sync_copy(col_hbm.at[pl.ds(ebase, EPT_G)], call)

    hr_b = (hr0, hr1)
    hc_b = (hc0, hc1)
    semr = (semr0, semr1)
    semc = (semc0, semc1)

    def gather(k, b):
        idx_r = rall.at[pl.ds(k * CS, CS)]
        idx_c = call.at[pl.ds(k * CS, CS)]
        pltpu.async_copy(h_hbm.at[idx_r], hr_b[b], semr[b])
        pltpu.async_copy(h_hbm.at[idx_c], hc_b[b], semc[b])

    def wait(k, b):
        idx_r = rall.at[pl.ds(k * CS, CS)]
        idx_c = call.at[pl.ds(k * CS, CS)]
        pltpu.make_async_copy(h_hbm.at[idx_r], hr_b[b], semr[b]).wait()
        pltpu.make_async_copy(h_hbm.at[idx_c], hc_b[b], semc[b]).wait()

    def compute(k, b):
        hr = hr_b[b]
        hc = hc_b[b]
        for gi in range((CS + 15) // 16):
            nv = min(16, CS - gi * 16)

            def dot_body(i, outv):
                e = gi * 16 + i
                accv = jnp.zeros((16,), f32)
                for u in range(D_H // 32):
                    sl = pl.ds(u * 16, 16)
                    ar = plsc.bitcast(hr[e, sl], jnp.bfloat16)
                    ac = plsc.bitcast(hc[e, sl], jnp.bfloat16)
                    p0, p1 = plsc.unpack(
                        ar * ac, format=plsc.PackFormat.INTERLEAVED)
                    accv = accv + p0 + p1
                d = jnp.sum(accv)
                return jnp.where(iota16 == i, d, outv)
            outv = lax.fori_loop(0, nv, dot_body, jnp.zeros((16,), f32))
            sv[pl.ds(gi * 16, 16)] = 1.0 / (1.0 + jnp.exp(-outv))
        pltpu.sync_copy(sv.at[pl.ds(0, CS)],
                        out_hbm.at[pl.ds(ebase + k * CS, CS)])

    gather(0, 0)

    def pair_body(kk, _):
        a = 2 * kk
        gather(a + 1, 1)
        wait(a, 0)
        compute(a, 0)
        gather(a + 2, 0)
        wait(a + 1, 1)
        compute(a + 1, 1)
        return 0
    lax.fori_loop(0, (NCS - 1) // 2, pair_body, 0)

    wait(NCS - 1, 0)
    compute(NCS - 1, 0)


def _k1_body(x_ref, w1_ref, d0_ref, d1_ref, out_ref):
    deg = 1.0 + d0_ref[...] + d1_ref[...]
    dinv = lax.rsqrt(deg)
    h = jnp.dot(x_ref[...], w1_ref[...], preferred_element_type=f32)
    out_ref[...] = h * dinv


def _k1(x, W1, d0, d1):
    return pl.pallas_call(
        _k1_body,
        grid=(NRB, NFC),
        in_specs=[
            pl.BlockSpec((RB, D_IN), lambda i, f: (i, 0)),
            pl.BlockSpec((D_IN, FC), lambda i, f: (0, f)),
            pl.BlockSpec((RB, 1), lambda i, f: (i, 0)),
            pl.BlockSpec((RB, 1), lambda i, f: (i, 0)),
        ],
        out_specs=pl.BlockSpec((RB, FC), lambda i, f: (f * NRB + i, 0)),
        out_shape=jax.ShapeDtypeStruct((NFC * N, FC), f32),
    )(x, W1, d0, d1)


def _k2_body(s10, s11, s12, s13, g10, g11, g12, g13, d0_ref, d1_ref,
             w2_ref, b1_ref, out_ref):
    deg = 1.0 + d0_ref[...] + d1_ref[...]
    dinv = lax.rsqrt(deg)
    s_k = (s10, s11, s12, s13)
    g_k = (g10, g11, g12, g13)
    z = jnp.concatenate(
        [jnp.maximum(dinv * (s_k[k][...] + g_k[k][...])
                     + b1_ref[k, 0][None, :], 0.0)
         for k in range(NFC)], axis=1).astype(jnp.bfloat16)
    out_ref[...] = jnp.dot(z, w2_ref[...], preferred_element_type=f32) * dinv


def _k2(s1, g1, d0, d1, W2, b1r):
    def chunk_spec(k):
        return pl.BlockSpec((RB, FC), lambda i, f, k=k: (k * NRB + i, 0))

    return pl.pallas_call(
        _k2_body,
        grid=(NRB, NFC),
        in_specs=(
            [chunk_spec(k) for k in range(NFC)] * 2
            + [
                pl.BlockSpec((RB, 1), lambda i, f: (i, 0)),
                pl.BlockSpec((RB, 1), lambda i, f: (i, 0)),
                pl.BlockSpec((D_H, FC), lambda i, f: (0, f)),
                pl.BlockSpec((NFC, 1, FC), lambda i, f: (0, 0, 0)),
            ]
        ),
        out_specs=pl.BlockSpec((RB, FC), lambda i, f: (f * NRB + i, 0)),
        out_shape=jax.ShapeDtypeStruct((NFC * N, FC), f32),
        compiler_params=pltpu.CompilerParams(
            dimension_semantics=("parallel", "parallel")),
    )(s1, s1, s1, s1, g1, g1, g1, g1, d0, d1, W2, b1r)


def _k3_body(s20, s21, s22, s23, g20, g21, g22, g23, d0_ref, d1_ref,
             b2_ref, out_ref):
    deg = 1.0 + d0_ref[...] + d1_ref[...]
    dinv = lax.rsqrt(deg)
    s_k = (s20, s21, s22, s23)
    g_k = (g20, g21, g22, g23)
    parts = [(dinv * (s_k[k][...] + g_k[k][...]) + b2_ref[k, 0][None, :]
              ).astype(jnp.bfloat16)
             for k in range(NFC)]
    ha = jnp.concatenate(parts[:2], axis=1)
    hb = jnp.concatenate(parts[2:], axis=1)
    wa = lax.bitcast_convert_type(ha, jnp.uint16).astype(i32)
    wb = lax.bitcast_convert_type(hb, jnp.uint16).astype(i32)
    out_ref[...] = wa | (wb << 16)


def _k3(s2, g2, d0, d1, b2r):
    def chunk_spec(k):
        return pl.BlockSpec((RB, FC), lambda i, k=k: (k * NRB + i, 0))

    return pl.pallas_call(
        _k3_body,
        grid=(NRB,),
        in_specs=(
            [chunk_spec(k) for k in range(NFC)] * 2
            + [
                pl.BlockSpec((RB, 1), lambda i: (i, 0)),
                pl.BlockSpec((RB, 1), lambda i: (i, 0)),
                pl.BlockSpec((NFC, 1, FC), lambda i: (0, 0, 0)),
            ]
        ),
        out_specs=pl.BlockSpec((RB, D_H // 2), lambda i: (i, 0)),
        out_shape=jax.ShapeDtypeStruct((N, D_H // 2), i32),
    )(s2, s2, s2, s2, g2, g2, g2, g2, d0, d1, b2r)


def kernel(x, edge_index, edge_weight, W1, b1, W2, b2):
    row = edge_index[0].astype(i32)
    col = edge_index[1].astype(i32)
    ew = edge_weight.astype(f32)
    b1r = b1.reshape(NFC, 1, FC)
    b2r = b2.reshape(NFC, 1, FC)
    xb = x.astype(jnp.bfloat16)
    W1b = W1.astype(jnp.bfloat16)
    W2b = W2.astype(jnp.bfloat16)

    deg_part = _deg_kernel(col, ew)
    d0 = deg_part[0, :N].reshape(N, 1)
    d1 = deg_part[1, :N].reshape(N, 1)
    g1 = _k1(xb, W1b, d0, d1)
    s1 = _prop_kernel(g1, row, col, ew)
    g2 = _k2(s1, g1, d0, d1, W2b, b1r)
    s2 = _prop_kernel(g2, row, col, ew)
    h = _k3(s2, g2, d0, d1, b2r)
    return _score_kernel(h, row, col)
